# Initial kernel scaffold; baseline (speedup 1.0000x reference)
import jax
import jax.numpy as jnp
from jax.experimental import pallas as pl


def kernel(x, edge_index, chosen_wordemb, leave_inds, W1, b1, W2, b2, Ws1, bs1, Ws2, bs2):
    raise NotImplementedError("write your pallas kernel here")


# speedup vs baseline: 4.1644x; 4.1644x over previous
import functools

import jax
import jax.numpy as jnp
from jax import lax
from jax.experimental import pallas as pl
from jax.experimental.pallas import tpu as pltpu
from jax.experimental.pallas import tpu_sc as plsc

N = 10000
E = 160000
D = 256
HD = 128
NL = 5000
NLP = 5120
NP = 10240
NC = 2
NS = 16
EK = 40

_SC_MESH = dict(core_axis_name="c", subcore_axis_name="s",
                num_cores=NC, num_subcores=NS)


def _sc_deg_body(dst_hbm, degp_hbm, dstv, onesv, bounce, deg_sh):
    s = lax.axis_index("s")
    c = lax.axis_index("c")

    def _zero16(i, carry):
        bounce[pl.ds(i * 16, 16)] = jnp.zeros((16,), jnp.float32)
        return carry

    lax.fori_loop(0, 640 // 16, _zero16, 0)
    pltpu.sync_copy(bounce, deg_sh.at[pl.ds(s * 640, 640)])
    plsc.subcore_barrier()

    onesv[pl.ds(0, 16)] = jnp.ones((16,), jnp.float32)
    onesv[pl.ds(16, 16)] = jnp.ones((16,), jnp.float32)
    onesv[pl.ds(EK - 16, 16)] = jnp.ones((16,), jnp.float32)

    per_tile = E // NC // NS
    base0 = c * (E // NC) + s * per_tile

    def _step(j, carry):
        base = base0 + j * EK
        pltpu.sync_copy(dst_hbm.at[pl.ds(base, EK)], dstv)
        pltpu.sync_copy(onesv, deg_sh.at[dstv], add=True)
        return carry

    lax.fori_loop(0, per_tile // EK, _step, 0)
    plsc.subcore_barrier()

    pltpu.sync_copy(deg_sh.at[pl.ds(s * 640, 640)], bounce)
    pltpu.sync_copy(bounce, degp_hbm.at[c, pl.ds(s * 640, 640)])


_deg_call = pl.kernel(
    _sc_deg_body,
    out_type=jax.ShapeDtypeStruct((NC, NP), jnp.float32),
    mesh=plsc.VectorSubcoreMesh(**_SC_MESH),
    scratch_types=[
        pltpu.VMEM((EK,), jnp.int32),
        pltpu.VMEM((EK,), jnp.float32),
        pltpu.VMEM((640,), jnp.float32),
        pltpu.VMEM_SHARED((NP,), jnp.float32),
    ],
)


def _sc_agg_body(hd_hbm, src_hbm, dst_hbm, S_hbm, sidx, didx, rows, wb, sem,
                 acc_sh):
    s = lax.axis_index("s")
    c = lax.axis_index("c")

    def _zero16(i, carry):
        wb[i // 8, pl.ds((i % 8) * 16, 16)] = jnp.zeros((16,), jnp.float32)
        return carry

    lax.fori_loop(0, 128, _zero16, 0)

    def _zshared(j, carry):
        pltpu.sync_copy(wb, acc_sh.at[pl.ds(s * 640 + j * 16, 16)])
        return carry

    lax.fori_loop(0, 640 // 16, _zshared, 0)
    plsc.subcore_barrier()

    per_tile = E // NS
    base0 = s * per_tile

    def _step(j, carry):
        base = base0 + j * EK
        pltpu.sync_copy(src_hbm.at[pl.ds(base, EK)], sidx)
        pltpu.sync_copy(dst_hbm.at[pl.ds(base, EK)], didx)
        pltpu.async_copy(hd_hbm.at[c].at[sidx], rows, sem).wait()
        pltpu.sync_copy(rows, acc_sh.at[didx], add=True)
        return carry

    lax.fori_loop(0, per_tile // EK, _step, 0)
    plsc.subcore_barrier()

    def _out(j, carry):
        r0 = s * 640 + j * 16
        pltpu.sync_copy(acc_sh.at[pl.ds(r0, 16)], wb)
        pltpu.sync_copy(wb, S_hbm.at[c].at[pl.ds(r0, 16)])
        return carry

    lax.fori_loop(0, 640 // 16, _out, 0)


_agg_call = pl.kernel(
    _sc_agg_body,
    out_type=jax.ShapeDtypeStruct((NC, NP, HD), jnp.float32),
    mesh=plsc.VectorSubcoreMesh(**_SC_MESH),
    scratch_types=[
        pltpu.VMEM((EK,), jnp.int32),
        pltpu.VMEM((EK,), jnp.int32),
        pltpu.VMEM((EK, HD), jnp.float32),
        pltpu.VMEM((16, HD), jnp.float32),
        pltpu.SemaphoreType.DMA,
        pltpu.VMEM_SHARED((NP, HD), jnp.float32),
    ],
)


def _sc_leaf_body(h2_hbm, li_hbm, out_hbm, lidx, lrows, sem):
    s = lax.axis_index("s")
    c = lax.axis_index("c")
    w = s * NC + c

    def _step(j, carry):
        base = w * 160 + j * 80
        pltpu.sync_copy(li_hbm.at[pl.ds(base, 80)], lidx)
        pltpu.async_copy(h2_hbm.at[lidx], lrows, sem).wait()
        pltpu.sync_copy(lrows, out_hbm.at[pl.ds(base, 80)])
        return carry

    lax.fori_loop(0, 2, _step, 0)


_leaf_call = pl.kernel(
    _sc_leaf_body,
    out_type=jax.ShapeDtypeStruct((NLP, D), jnp.float32),
    mesh=plsc.VectorSubcoreMesh(**_SC_MESH),
    scratch_types=[
        pltpu.VMEM((80,), jnp.int32),
        pltpu.VMEM((80, D), jnp.float32),
        pltpu.SemaphoreType.DMA,
    ],
)


_RB = 2000


def _tc_a_body(degp_ref, x_ref, w1_ref, hw_ref, hd_ref, dinv_ref, dinv2_ref):
    deg = degp_ref[0] + degp_ref[1] + 1.0
    dinv = lax.rsqrt(deg)
    dinv2 = dinv * dinv
    hw = jnp.dot(x_ref[...], w1_ref[...], preferred_element_type=jnp.float32)
    hw_ref[...] = hw
    hd_ref[0] = hw[:, :HD] * dinv
    hd_ref[1] = hw[:, HD:] * dinv
    dinv_ref[...] = dinv
    dinv2_ref[...] = dinv2


def _stage_a(degp, x, W1):
    return pl.pallas_call(
        _tc_a_body,
        grid=(N // _RB,),
        in_specs=[
            pl.BlockSpec((NC, _RB, 1), lambda i: (0, i, 0)),
            pl.BlockSpec((_RB, D), lambda i: (i, 0)),
            pl.BlockSpec((D, D), lambda i: (0, 0)),
        ],
        out_specs=[
            pl.BlockSpec((_RB, D), lambda i: (i, 0)),
            pl.BlockSpec((NC, _RB, HD), lambda i: (0, i, 0)),
            pl.BlockSpec((_RB, 1), lambda i: (i, 0)),
            pl.BlockSpec((_RB, 1), lambda i: (i, 0)),
        ],
        out_shape=[
            jax.ShapeDtypeStruct((N, D), jnp.float32),
            jax.ShapeDtypeStruct((NC, N, HD), jnp.float32),
            jax.ShapeDtypeStruct((N, 1), jnp.float32),
            jax.ShapeDtypeStruct((N, 1), jnp.float32),
        ],
    )(degp, x, W1)


def _tc_b_body(S_ref, hw_ref, dinv_ref, dinv2_ref, b_ref, w2_ref,
               hw2_ref, hd2_ref):
    Sc = jnp.concatenate([S_ref[0], S_ref[1]], axis=1)
    dinv = dinv_ref[...]
    h1 = Sc * dinv + hw_ref[...] * dinv2_ref[...] + b_ref[...]
    h1 = jnp.maximum(h1, 0.0)
    hw2 = jnp.dot(h1, w2_ref[...], preferred_element_type=jnp.float32)
    hw2_ref[...] = hw2
    hd2_ref[0] = hw2[:, :HD] * dinv
    hd2_ref[1] = hw2[:, HD:] * dinv


def _stage_b(S1, hw1, dinv, dinv2, b1, W2):
    return pl.pallas_call(
        _tc_b_body,
        grid=(N // _RB,),
        in_specs=[
            pl.BlockSpec((NC, _RB, HD), lambda i: (0, i, 0)),
            pl.BlockSpec((_RB, D), lambda i: (i, 0)),
            pl.BlockSpec((_RB, 1), lambda i: (i, 0)),
            pl.BlockSpec((_RB, 1), lambda i: (i, 0)),
            pl.BlockSpec((1, D), lambda i: (0, 0)),
            pl.BlockSpec((D, D), lambda i: (0, 0)),
        ],
        out_specs=[
            pl.BlockSpec((_RB, D), lambda i: (i, 0)),
            pl.BlockSpec((NC, _RB, HD), lambda i: (0, i, 0)),
        ],
        out_shape=[
            jax.ShapeDtypeStruct((N, D), jnp.float32),
            jax.ShapeDtypeStruct((NC, N, HD), jnp.float32),
        ],
    )(S1, hw1, dinv, dinv2, b1, W2)


def _tc_c_body(S_ref, hw_ref, dinv_ref, dinv2_ref, b_ref, h2_ref, cs_ref):
    i = pl.program_id(0)
    Sc = jnp.concatenate([S_ref[0], S_ref[1]], axis=1)
    h2 = Sc * dinv_ref[...] + hw_ref[...] * dinv2_ref[...] + b_ref[...]
    h2_ref[...] = h2

    @pl.when(i == 0)
    def _():
        cs_ref[...] = jnp.zeros_like(cs_ref)

    cs_ref[...] += jnp.sum(h2, axis=0, keepdims=True)


def _stage_c(S2, hw2, dinv, dinv2, b2):
    return pl.pallas_call(
        _tc_c_body,
        grid=(N // _RB,),
        in_specs=[
            pl.BlockSpec((NC, _RB, HD), lambda i: (0, i, 0)),
            pl.BlockSpec((_RB, D), lambda i: (i, 0)),
            pl.BlockSpec((_RB, 1), lambda i: (i, 0)),
            pl.BlockSpec((_RB, 1), lambda i: (i, 0)),
            pl.BlockSpec((1, D), lambda i: (0, 0)),
        ],
        out_specs=[
            pl.BlockSpec((_RB, D), lambda i: (i, 0)),
            pl.BlockSpec((1, D), lambda i: (0, 0)),
        ],
        out_shape=[
            jax.ShapeDtypeStruct((N, D), jnp.float32),
            jax.ShapeDtypeStruct((1, D), jnp.float32),
        ],
    )(S2, hw2, dinv, dinv2, b2)


def _tc_d_body(g_ref, cs_ref, cw_ref, wa_ref, wh_ref, wc_ref, bs1_ref,
               ws2_ref, bs2_ref, out_ref):
    gh = cs_ref[...] * (1.0 / N)
    c0 = (jnp.dot(cw_ref[...], wa_ref[...], preferred_element_type=jnp.float32)
          + jnp.dot(gh, wc_ref[...], preferred_element_type=jnp.float32)
          + bs1_ref[...])
    sact = jnp.dot(g_ref[...], wh_ref[...], preferred_element_type=jnp.float32)
    sact = jnp.maximum(sact + c0, 0.0)
    logits = (jnp.dot(sact, ws2_ref[...], preferred_element_type=jnp.float32)
              + bs2_ref[...])
    rows = lax.broadcasted_iota(jnp.int32, (NLP, 1), 0)
    logits = jnp.where(rows < NL, logits, -1e30)
    m = jnp.max(logits)
    p = jnp.exp(logits - m)
    out_ref[...] = p / jnp.sum(p)


def _stage_d(g, colsum, cw, Wa, Wh, Wc, bs1, Ws2, bs2):
    return pl.pallas_call(
        _tc_d_body,
        out_shape=jax.ShapeDtypeStruct((NLP, 1), jnp.float32),
    )(g, colsum, cw, Wa, Wh, Wc, bs1, Ws2, bs2)


def kernel(x, edge_index, chosen_wordemb, leave_inds, W1, b1, W2, b2,
           Ws1, bs1, Ws2, bs2):
    src = edge_index[0].astype(jnp.int32)
    dst = edge_index[1].astype(jnp.int32)
    li = jnp.pad(leave_inds.astype(jnp.int32), (0, NLP - NL))

    degp = _deg_call(dst)
    degp3 = degp[:, :N, None]

    hw1, hd1, dinv, dinv2 = _stage_a(degp3, x, W1)
    S1 = _agg_call(hd1, src, dst)
    hw2, hd2 = _stage_b(S1[:, :N], hw1, dinv, dinv2, b1.reshape(1, D), W2)
    S2 = _agg_call(hd2, src, dst)
    h2, colsum = _stage_c(S2[:, :N], hw2, dinv, dinv2, b2.reshape(1, D))

    g = _leaf_call(h2, li)

    cw = chosen_wordemb.reshape(1, D)
    Wa, Wh, Wc = Ws1[:D], Ws1[D:2 * D], Ws1[2 * D:]
    scores = _stage_d(g, colsum, cw, Wa, Wh, Wc,
                      bs1.reshape(1, D), Ws2, bs2.reshape(1, 1))
    return scores.reshape(-1)[:NL]


# speedup vs baseline: 8.8369x; 2.1220x over previous
import functools

import jax
import jax.numpy as jnp
from jax import lax
from jax.experimental import pallas as pl
from jax.experimental.pallas import tpu as pltpu
from jax.experimental.pallas import tpu_sc as plsc

N = 10000
E = 160000
D = 256
HD = 128
NL = 5000
NLP = 5120
NP = 10240
NC = 2
NS = 16
EK = 40

_SC_MESH = dict(core_axis_name="c", subcore_axis_name="s",
                num_cores=NC, num_subcores=NS)


def _sc_deg_body(dst_hbm, degp_hbm, dstv, onesv, bounce, deg_sh):
    s = lax.axis_index("s")
    c = lax.axis_index("c")

    def _zero16(i, carry):
        bounce[pl.ds(i * 16, 16)] = jnp.zeros((16,), jnp.float32)
        return carry

    lax.fori_loop(0, 640 // 16, _zero16, 0)
    pltpu.sync_copy(bounce, deg_sh.at[pl.ds(s * 640, 640)])
    plsc.subcore_barrier()

    onesv[pl.ds(0, 16)] = jnp.ones((16,), jnp.float32)
    onesv[pl.ds(16, 16)] = jnp.ones((16,), jnp.float32)
    onesv[pl.ds(EK - 16, 16)] = jnp.ones((16,), jnp.float32)

    per_tile = E // NC // NS
    base0 = c * (E // NC) + s * per_tile

    def _step(j, carry):
        base = base0 + j * EK
        pltpu.sync_copy(dst_hbm.at[pl.ds(base, EK)], dstv)
        pltpu.sync_copy(onesv, deg_sh.at[dstv], add=True)
        return carry

    lax.fori_loop(0, per_tile // EK, _step, 0)
    plsc.subcore_barrier()

    pltpu.sync_copy(deg_sh.at[pl.ds(s * 640, 640)], bounce)
    pltpu.sync_copy(bounce, degp_hbm.at[c, pl.ds(s * 640, 640)])


_deg_call = pl.kernel(
    _sc_deg_body,
    out_type=jax.ShapeDtypeStruct((NC, NP), jnp.float32),
    mesh=plsc.VectorSubcoreMesh(**_SC_MESH),
    scratch_types=[
        pltpu.VMEM((EK,), jnp.int32),
        pltpu.VMEM((EK,), jnp.float32),
        pltpu.VMEM((640,), jnp.float32),
        pltpu.VMEM_SHARED((NP,), jnp.float32),
    ],
)


EKA = 80


def _sc_agg_body(hd_hbm, src_hbm, dst_hbm, S_hbm,
                 sidx0, sidx1, didx0, didx1, rows0, rows1, wb,
                 sem0, sem1, acc_sh):
    s = lax.axis_index("s")
    c = lax.axis_index("c")
    sidx = (sidx0, sidx1)
    didx = (didx0, didx1)
    rows = (rows0, rows1)
    sems = (sem0, sem1)

    def _zero16(i, carry):
        wb[i // 8, pl.ds((i % 8) * 16, 16)] = jnp.zeros((16,), jnp.float32)
        return carry

    lax.fori_loop(0, 128, _zero16, 0)

    def _zshared(j, carry):
        pltpu.sync_copy(wb, acc_sh.at[pl.ds(s * 640 + j * 16, 16)])
        return carry

    lax.fori_loop(0, 640 // 16, _zshared, 0)
    plsc.subcore_barrier()

    per_tile = E // NS
    nch = per_tile // EKA
    base0 = s * per_tile

    pltpu.sync_copy(src_hbm.at[pl.ds(base0, EKA)], sidx0)
    pltpu.sync_copy(dst_hbm.at[pl.ds(base0, EKA)], didx0)
    pltpu.async_copy(hd_hbm.at[c].at[sidx0], rows0, sem0)

    def _pair(i, carry):
        for b in range(2):
            j = i * 2 + b
            nxt = 1 - b
            basen = base0 + (j + 1) * EKA
            pltpu.sync_copy(src_hbm.at[pl.ds(basen, EKA)], sidx[nxt])
            pltpu.sync_copy(dst_hbm.at[pl.ds(basen, EKA)], didx[nxt])
            pltpu.async_copy(hd_hbm.at[c].at[sidx[nxt]], rows[nxt], sems[nxt])
            pltpu.make_async_copy(hd_hbm.at[c].at[sidx[b]], rows[b],
                                  sems[b]).wait()
            pltpu.sync_copy(rows[b], acc_sh.at[didx[b]], add=True)
        return carry

    lax.fori_loop(0, (nch - 1) // 2, _pair, 0)
    pltpu.make_async_copy(hd_hbm.at[c].at[sidx0], rows0, sem0).wait()
    pltpu.sync_copy(rows0, acc_sh.at[didx0], add=True)
    plsc.subcore_barrier()

    def _out(j, carry):
        r0 = s * 640 + j * 16
        pltpu.sync_copy(acc_sh.at[pl.ds(r0, 16)], wb)
        pltpu.sync_copy(wb, S_hbm.at[c].at[pl.ds(r0, 16)])
        return carry

    lax.fori_loop(0, 640 // 16, _out, 0)


_agg_call = pl.kernel(
    _sc_agg_body,
    out_type=jax.ShapeDtypeStruct((NC, NP, HD), jnp.float32),
    mesh=plsc.VectorSubcoreMesh(**_SC_MESH),
    scratch_types=[
        pltpu.VMEM((EKA,), jnp.int32),
        pltpu.VMEM((EKA,), jnp.int32),
        pltpu.VMEM((EKA,), jnp.int32),
        pltpu.VMEM((EKA,), jnp.int32),
        pltpu.VMEM((EKA, HD), jnp.float32),
        pltpu.VMEM((EKA, HD), jnp.float32),
        pltpu.VMEM((16, HD), jnp.float32),
        pltpu.SemaphoreType.DMA,
        pltpu.SemaphoreType.DMA,
        pltpu.VMEM_SHARED((NP, HD), jnp.float32),
    ],
)


def _sc_leaf_body(h2_hbm, li_hbm, out_hbm, lidx, lrows, sem):
    s = lax.axis_index("s")
    c = lax.axis_index("c")
    w = s * NC + c

    def _step(j, carry):
        base = w * 160 + j * 80
        pltpu.sync_copy(li_hbm.at[pl.ds(base, 80)], lidx)
        pltpu.async_copy(h2_hbm.at[lidx], lrows, sem).wait()
        pltpu.sync_copy(lrows, out_hbm.at[pl.ds(base, 80)])
        return carry

    lax.fori_loop(0, 2, _step, 0)


_leaf_call = pl.kernel(
    _sc_leaf_body,
    out_type=jax.ShapeDtypeStruct((NLP, D), jnp.float32),
    mesh=plsc.VectorSubcoreMesh(**_SC_MESH),
    scratch_types=[
        pltpu.VMEM((80,), jnp.int32),
        pltpu.VMEM((80, D), jnp.float32),
        pltpu.SemaphoreType.DMA,
    ],
)


_RB = 2000


def _tc_a_body(degp_ref, x_ref, w1_ref, hw_ref, hd_ref, dinv_ref, dinv2_ref):
    deg = degp_ref[0] + degp_ref[1] + 1.0
    dinv = lax.rsqrt(deg)
    dinv2 = dinv * dinv
    hw = jnp.dot(x_ref[...], w1_ref[...], preferred_element_type=jnp.float32)
    hw_ref[...] = hw
    hd_ref[0] = hw[:, :HD] * dinv
    hd_ref[1] = hw[:, HD:] * dinv
    dinv_ref[...] = dinv
    dinv2_ref[...] = dinv2


def _stage_a(degp, x, W1):
    return pl.pallas_call(
        _tc_a_body,
        grid=(N // _RB,),
        in_specs=[
            pl.BlockSpec((NC, _RB, 1), lambda i: (0, i, 0)),
            pl.BlockSpec((_RB, D), lambda i: (i, 0)),
            pl.BlockSpec((D, D), lambda i: (0, 0)),
        ],
        out_specs=[
            pl.BlockSpec((_RB, D), lambda i: (i, 0)),
            pl.BlockSpec((NC, _RB, HD), lambda i: (0, i, 0)),
            pl.BlockSpec((_RB, 1), lambda i: (i, 0)),
            pl.BlockSpec((_RB, 1), lambda i: (i, 0)),
        ],
        out_shape=[
            jax.ShapeDtypeStruct((N, D), jnp.float32),
            jax.ShapeDtypeStruct((NC, N, HD), jnp.float32),
            jax.ShapeDtypeStruct((N, 1), jnp.float32),
            jax.ShapeDtypeStruct((N, 1), jnp.float32),
        ],
    )(degp, x, W1)


def _tc_b_body(S_ref, hw_ref, dinv_ref, dinv2_ref, b_ref, w2_ref,
               hw2_ref, hd2_ref):
    Sc = jnp.concatenate([S_ref[0], S_ref[1]], axis=1)
    dinv = dinv_ref[...]
    h1 = Sc * dinv + hw_ref[...] * dinv2_ref[...] + b_ref[...]
    h1 = jnp.maximum(h1, 0.0)
    hw2 = jnp.dot(h1, w2_ref[...], preferred_element_type=jnp.float32)
    hw2_ref[...] = hw2
    hd2_ref[0] = hw2[:, :HD] * dinv
    hd2_ref[1] = hw2[:, HD:] * dinv


def _stage_b(S1, hw1, dinv, dinv2, b1, W2):
    return pl.pallas_call(
        _tc_b_body,
        grid=(N // _RB,),
        in_specs=[
            pl.BlockSpec((NC, _RB, HD), lambda i: (0, i, 0)),
            pl.BlockSpec((_RB, D), lambda i: (i, 0)),
            pl.BlockSpec((_RB, 1), lambda i: (i, 0)),
            pl.BlockSpec((_RB, 1), lambda i: (i, 0)),
            pl.BlockSpec((1, D), lambda i: (0, 0)),
            pl.BlockSpec((D, D), lambda i: (0, 0)),
        ],
        out_specs=[
            pl.BlockSpec((_RB, D), lambda i: (i, 0)),
            pl.BlockSpec((NC, _RB, HD), lambda i: (0, i, 0)),
        ],
        out_shape=[
            jax.ShapeDtypeStruct((N, D), jnp.float32),
            jax.ShapeDtypeStruct((NC, N, HD), jnp.float32),
        ],
    )(S1, hw1, dinv, dinv2, b1, W2)


def _tc_c_body(S_ref, hw_ref, dinv_ref, dinv2_ref, b_ref, h2_ref, cs_ref):
    i = pl.program_id(0)
    Sc = jnp.concatenate([S_ref[0], S_ref[1]], axis=1)
    h2 = Sc * dinv_ref[...] + hw_ref[...] * dinv2_ref[...] + b_ref[...]
    h2_ref[...] = h2

    @pl.when(i == 0)
    def _():
        cs_ref[...] = jnp.zeros_like(cs_ref)

    cs_ref[...] += jnp.sum(h2, axis=0, keepdims=True)


def _stage_c(S2, hw2, dinv, dinv2, b2):
    return pl.pallas_call(
        _tc_c_body,
        grid=(N // _RB,),
        in_specs=[
            pl.BlockSpec((NC, _RB, HD), lambda i: (0, i, 0)),
            pl.BlockSpec((_RB, D), lambda i: (i, 0)),
            pl.BlockSpec((_RB, 1), lambda i: (i, 0)),
            pl.BlockSpec((_RB, 1), lambda i: (i, 0)),
            pl.BlockSpec((1, D), lambda i: (0, 0)),
        ],
        out_specs=[
            pl.BlockSpec((_RB, D), lambda i: (i, 0)),
            pl.BlockSpec((1, D), lambda i: (0, 0)),
        ],
        out_shape=[
            jax.ShapeDtypeStruct((N, D), jnp.float32),
            jax.ShapeDtypeStruct((1, D), jnp.float32),
        ],
    )(S2, hw2, dinv, dinv2, b2)


def _tc_d_body(g_ref, cs_ref, cw_ref, wa_ref, wh_ref, wc_ref, bs1_ref,
               ws2_ref, bs2_ref, out_ref):
    gh = cs_ref[...] * (1.0 / N)
    c0 = (jnp.dot(cw_ref[...], wa_ref[...], preferred_element_type=jnp.float32)
          + jnp.dot(gh, wc_ref[...], preferred_element_type=jnp.float32)
          + bs1_ref[...])
    sact = jnp.dot(g_ref[...], wh_ref[...], preferred_element_type=jnp.float32)
    sact = jnp.maximum(sact + c0, 0.0)
    logits = (jnp.dot(sact, ws2_ref[...], preferred_element_type=jnp.float32)
              + bs2_ref[...])
    rows = lax.broadcasted_iota(jnp.int32, (NLP, 1), 0)
    logits = jnp.where(rows < NL, logits, -1e30)
    m = jnp.max(logits)
    p = jnp.exp(logits - m)
    out_ref[...] = p / jnp.sum(p)


def _stage_d(g, colsum, cw, Wa, Wh, Wc, bs1, Ws2, bs2):
    return pl.pallas_call(
        _tc_d_body,
        out_shape=jax.ShapeDtypeStruct((NLP, 1), jnp.float32),
    )(g, colsum, cw, Wa, Wh, Wc, bs1, Ws2, bs2)


def kernel(x, edge_index, chosen_wordemb, leave_inds, W1, b1, W2, b2,
           Ws1, bs1, Ws2, bs2):
    src = edge_index[0].astype(jnp.int32)
    dst = edge_index[1].astype(jnp.int32)
    li = jnp.pad(leave_inds.astype(jnp.int32), (0, NLP - NL))

    degp = _deg_call(dst)
    degp3 = degp[:, :N, None]

    hw1, hd1, dinv, dinv2 = _stage_a(degp3, x, W1)
    S1 = _agg_call(hd1, src, dst)
    hw2, hd2 = _stage_b(S1[:, :N], hw1, dinv, dinv2, b1.reshape(1, D), W2)
    S2 = _agg_call(hd2, src, dst)
    h2, colsum = _stage_c(S2[:, :N], hw2, dinv, dinv2, b2.reshape(1, D))

    g = _leaf_call(h2, li)

    cw = chosen_wordemb.reshape(1, D)
    Wa, Wh, Wc = Ws1[:D], Ws1[D:2 * D], Ws1[2 * D:]
    scores = _stage_d(g, colsum, cw, Wa, Wh, Wc,
                      bs1.reshape(1, D), Ws2, bs2.reshape(1, 1))
    return scores.reshape(-1)[:NL]


# speedup vs baseline: 11.3058x; 1.2794x over previous
import functools

import jax
import jax.numpy as jnp
from jax import lax
from jax.experimental import pallas as pl
from jax.experimental.pallas import tpu as pltpu
from jax.experimental.pallas import tpu_sc as plsc

N = 10000
E = 160000
D = 256
HD = 128
NL = 5000
NLP = 5120
NP = 10240
NC = 2
NS = 16
EK = 40

_SC_MESH = dict(core_axis_name="c", subcore_axis_name="s",
                num_cores=NC, num_subcores=NS)


def _sc_deg_body(dst_hbm, degp_hbm, dstv, onesv, bounce, deg_sh):
    s = lax.axis_index("s")
    c = lax.axis_index("c")

    def _zero16(i, carry):
        bounce[pl.ds(i * 16, 16)] = jnp.zeros((16,), jnp.float32)
        return carry

    lax.fori_loop(0, 640 // 16, _zero16, 0)
    pltpu.sync_copy(bounce, deg_sh.at[pl.ds(s * 640, 640)])
    plsc.subcore_barrier()

    onesv[pl.ds(0, 16)] = jnp.ones((16,), jnp.float32)
    onesv[pl.ds(16, 16)] = jnp.ones((16,), jnp.float32)
    onesv[pl.ds(EK - 16, 16)] = jnp.ones((16,), jnp.float32)

    per_tile = E // NC // NS
    base0 = c * (E // NC) + s * per_tile

    def _step(j, carry):
        base = base0 + j * EK
        pltpu.sync_copy(dst_hbm.at[pl.ds(base, EK)], dstv)
        pltpu.sync_copy(onesv, deg_sh.at[dstv], add=True)
        return carry

    lax.fori_loop(0, per_tile // EK, _step, 0)
    plsc.subcore_barrier()

    pltpu.sync_copy(deg_sh.at[pl.ds(s * 640, 640)], bounce)
    pltpu.sync_copy(bounce, degp_hbm.at[c, pl.ds(s * 640, 640)])


_deg_call = pl.kernel(
    _sc_deg_body,
    out_type=jax.ShapeDtypeStruct((NC, NP), jnp.float32),
    mesh=plsc.VectorSubcoreMesh(**_SC_MESH),
    scratch_types=[
        pltpu.VMEM((EK,), jnp.int32),
        pltpu.VMEM((EK,), jnp.float32),
        pltpu.VMEM((640,), jnp.float32),
        pltpu.VMEM_SHARED((NP,), jnp.float32),
    ],
)


EKA = 80


_NCH = (E // NS) // EKA


_NSUP = 5
_SUP = _NCH // _NSUP


def _sc_agg_body(hd_hbm, src4_hbm, dst4_hbm, S_hbm,
                 sidx_m, didx_m, rows0, rows1, zb,
                 sem0, sem1, acc_sh):
    s = lax.axis_index("s")
    c = lax.axis_index("c")
    rows = (rows0, rows1)
    sems = (sem0, sem1)

    def _zero16(i, carry):
        zb[i // 8, pl.ds((i % 8) * 16, 16)] = jnp.zeros((16,), jnp.float32)
        return carry

    lax.fori_loop(0, 32 * 8, _zero16, 0)

    def _zshared(j, carry):
        pltpu.sync_copy(zb, acc_sh.at[pl.ds(s * 640 + j * 32, 32)])
        return carry

    lax.fori_loop(0, 640 // 32, _zshared, 0)
    plsc.subcore_barrier()

    def _super(g, carry):
        pltpu.sync_copy(src4_hbm.at[s, g], sidx_m)
        pltpu.sync_copy(dst4_hbm.at[s, g], didx_m)
        pltpu.async_copy(hd_hbm.at[c].at[sidx_m.at[0]], rows0, sem0)

        def _pair(i, carry2):
            for b in range(2):
                k = i * 2 + b
                nxt = 1 - b
                pltpu.async_copy(hd_hbm.at[c].at[sidx_m.at[k + 1]],
                                 rows[nxt], sems[nxt])
                pltpu.make_async_copy(hd_hbm.at[c].at[sidx_m.at[k]], rows[b],
                                      sems[b]).wait()
                pltpu.sync_copy(rows[b], acc_sh.at[didx_m.at[k]], add=True)
            return carry2

        lax.fori_loop(0, (_SUP - 1) // 2, _pair, 0)
        pltpu.make_async_copy(hd_hbm.at[c].at[sidx_m.at[_SUP - 1]], rows0,
                              sem0).wait()
        pltpu.sync_copy(rows0, acc_sh.at[didx_m.at[_SUP - 1]], add=True)
        return carry

    lax.fori_loop(0, _NSUP, _super, 0)
    plsc.subcore_barrier()

    pltpu.sync_copy(acc_sh.at[pl.ds(s * 640, 640)],
                    S_hbm.at[c].at[pl.ds(s * 640, 640)])


_agg_call = pl.kernel(
    _sc_agg_body,
    out_type=jax.ShapeDtypeStruct((NC, NP, HD), jnp.float32),
    mesh=plsc.VectorSubcoreMesh(**_SC_MESH),
    scratch_types=[
        pltpu.VMEM((_SUP, EKA), jnp.int32),
        pltpu.VMEM((_SUP, EKA), jnp.int32),
        pltpu.VMEM((EKA, HD), jnp.float32),
        pltpu.VMEM((EKA, HD), jnp.float32),
        pltpu.VMEM((32, HD), jnp.float32),
        pltpu.SemaphoreType.DMA,
        pltpu.SemaphoreType.DMA,
        pltpu.VMEM_SHARED((NP, HD), jnp.float32),
    ],
)


def _sc_leaf_body(h2_hbm, li_hbm, out_hbm, lidx, lrows, sem):
    s = lax.axis_index("s")
    c = lax.axis_index("c")
    w = s * NC + c

    def _step(j, carry):
        base = w * 160 + j * 80
        pltpu.sync_copy(li_hbm.at[pl.ds(base, 80)], lidx)
        pltpu.async_copy(h2_hbm.at[lidx], lrows, sem).wait()
        pltpu.sync_copy(lrows, out_hbm.at[pl.ds(base, 80)])
        return carry

    lax.fori_loop(0, 2, _step, 0)


_leaf_call = pl.kernel(
    _sc_leaf_body,
    out_type=jax.ShapeDtypeStruct((NLP, D), jnp.float32),
    mesh=plsc.VectorSubcoreMesh(**_SC_MESH),
    scratch_types=[
        pltpu.VMEM((80,), jnp.int32),
        pltpu.VMEM((80, D), jnp.float32),
        pltpu.SemaphoreType.DMA,
    ],
)


_RB = 2000


def _tc_a_body(degp_ref, x_ref, w1_ref, hw_ref, hd_ref, dinv_ref, dinv2_ref):
    deg = degp_ref[0] + degp_ref[1] + 1.0
    dinv = lax.rsqrt(deg)
    dinv2 = dinv * dinv
    hw = jnp.dot(x_ref[...], w1_ref[...], preferred_element_type=jnp.float32)
    hw_ref[...] = hw
    hd_ref[0] = hw[:, :HD] * dinv
    hd_ref[1] = hw[:, HD:] * dinv
    dinv_ref[...] = dinv
    dinv2_ref[...] = dinv2


def _stage_a(degp, x, W1):
    return pl.pallas_call(
        _tc_a_body,
        grid=(N // _RB,),
        in_specs=[
            pl.BlockSpec((NC, _RB, 1), lambda i: (0, i, 0)),
            pl.BlockSpec((_RB, D), lambda i: (i, 0)),
            pl.BlockSpec((D, D), lambda i: (0, 0)),
        ],
        out_specs=[
            pl.BlockSpec((_RB, D), lambda i: (i, 0)),
            pl.BlockSpec((NC, _RB, HD), lambda i: (0, i, 0)),
            pl.BlockSpec((_RB, 1), lambda i: (i, 0)),
            pl.BlockSpec((_RB, 1), lambda i: (i, 0)),
        ],
        out_shape=[
            jax.ShapeDtypeStruct((N, D), jnp.float32),
            jax.ShapeDtypeStruct((NC, N, HD), jnp.float32),
            jax.ShapeDtypeStruct((N, 1), jnp.float32),
            jax.ShapeDtypeStruct((N, 1), jnp.float32),
        ],
    )(degp, x, W1)


def _tc_b_body(S_ref, hw_ref, dinv_ref, dinv2_ref, b_ref, w2_ref,
               hw2_ref, hd2_ref):
    Sc = jnp.concatenate([S_ref[0], S_ref[1]], axis=1)
    dinv = dinv_ref[...]
    h1 = Sc * dinv + hw_ref[...] * dinv2_ref[...] + b_ref[...]
    h1 = jnp.maximum(h1, 0.0)
    hw2 = jnp.dot(h1, w2_ref[...], preferred_element_type=jnp.float32)
    hw2_ref[...] = hw2
    hd2_ref[0] = hw2[:, :HD] * dinv
    hd2_ref[1] = hw2[:, HD:] * dinv


def _stage_b(S1, hw1, dinv, dinv2, b1, W2):
    return pl.pallas_call(
        _tc_b_body,
        grid=(N // _RB,),
        in_specs=[
            pl.BlockSpec((NC, _RB, HD), lambda i: (0, i, 0)),
            pl.BlockSpec((_RB, D), lambda i: (i, 0)),
            pl.BlockSpec((_RB, 1), lambda i: (i, 0)),
            pl.BlockSpec((_RB, 1), lambda i: (i, 0)),
            pl.BlockSpec((1, D), lambda i: (0, 0)),
            pl.BlockSpec((D, D), lambda i: (0, 0)),
        ],
        out_specs=[
            pl.BlockSpec((_RB, D), lambda i: (i, 0)),
            pl.BlockSpec((NC, _RB, HD), lambda i: (0, i, 0)),
        ],
        out_shape=[
            jax.ShapeDtypeStruct((N, D), jnp.float32),
            jax.ShapeDtypeStruct((NC, N, HD), jnp.float32),
        ],
    )(S1, hw1, dinv, dinv2, b1, W2)


def _tc_c_body(S_ref, hw_ref, dinv_ref, dinv2_ref, b_ref, h2_ref, cs_ref):
    i = pl.program_id(0)
    Sc = jnp.concatenate([S_ref[0], S_ref[1]], axis=1)
    h2 = Sc * dinv_ref[...] + hw_ref[...] * dinv2_ref[...] + b_ref[...]
    h2_ref[...] = h2

    @pl.when(i == 0)
    def _():
        cs_ref[...] = jnp.zeros_like(cs_ref)

    cs_ref[...] += jnp.sum(h2, axis=0, keepdims=True)


def _stage_c(S2, hw2, dinv, dinv2, b2):
    return pl.pallas_call(
        _tc_c_body,
        grid=(N // _RB,),
        in_specs=[
            pl.BlockSpec((NC, _RB, HD), lambda i: (0, i, 0)),
            pl.BlockSpec((_RB, D), lambda i: (i, 0)),
            pl.BlockSpec((_RB, 1), lambda i: (i, 0)),
            pl.BlockSpec((_RB, 1), lambda i: (i, 0)),
            pl.BlockSpec((1, D), lambda i: (0, 0)),
        ],
        out_specs=[
            pl.BlockSpec((_RB, D), lambda i: (i, 0)),
            pl.BlockSpec((1, D), lambda i: (0, 0)),
        ],
        out_shape=[
            jax.ShapeDtypeStruct((N, D), jnp.float32),
            jax.ShapeDtypeStruct((1, D), jnp.float32),
        ],
    )(S2, hw2, dinv, dinv2, b2)


def _tc_d_body(g_ref, cs_ref, cw_ref, wa_ref, wh_ref, wc_ref, bs1_ref,
               ws2_ref, bs2_ref, out_ref):
    gh = cs_ref[...] * (1.0 / N)
    c0 = (jnp.dot(cw_ref[...], wa_ref[...], preferred_element_type=jnp.float32)
          + jnp.dot(gh, wc_ref[...], preferred_element_type=jnp.float32)
          + bs1_ref[...])
    sact = jnp.dot(g_ref[...], wh_ref[...], preferred_element_type=jnp.float32)
    sact = jnp.maximum(sact + c0, 0.0)
    logits = (jnp.dot(sact, ws2_ref[...], preferred_element_type=jnp.float32)
              + bs2_ref[...])
    rows = lax.broadcasted_iota(jnp.int32, (NLP, 1), 0)
    logits = jnp.where(rows < NL, logits, -1e30)
    m = jnp.max(logits)
    p = jnp.exp(logits - m)
    out_ref[...] = p / jnp.sum(p)


def _stage_d(g, colsum, cw, Wa, Wh, Wc, bs1, Ws2, bs2):
    return pl.pallas_call(
        _tc_d_body,
        out_shape=jax.ShapeDtypeStruct((NLP, 1), jnp.float32),
    )(g, colsum, cw, Wa, Wh, Wc, bs1, Ws2, bs2)


def kernel(x, edge_index, chosen_wordemb, leave_inds, W1, b1, W2, b2,
           Ws1, bs1, Ws2, bs2):
    src = edge_index[0].astype(jnp.int32)
    dst = edge_index[1].astype(jnp.int32)
    li = jnp.pad(leave_inds.astype(jnp.int32), (0, NLP - NL))

    degp = _deg_call(dst)
    degp3 = degp[:, :N, None]

    src3 = src.reshape(NS, _NSUP, _SUP, EKA)
    dst3 = dst.reshape(NS, _NSUP, _SUP, EKA)
    hw1, hd1, dinv, dinv2 = _stage_a(degp3, x, W1)
    S1 = _agg_call(hd1, src3, dst3)
    hw2, hd2 = _stage_b(S1[:, :N], hw1, dinv, dinv2, b1.reshape(1, D), W2)
    S2 = _agg_call(hd2, src3, dst3)
    h2, colsum = _stage_c(S2[:, :N], hw2, dinv, dinv2, b2.reshape(1, D))

    g = _leaf_call(h2, li)

    cw = chosen_wordemb.reshape(1, D)
    Wa, Wh, Wc = Ws1[:D], Ws1[D:2 * D], Ws1[2 * D:]
    scores = _stage_d(g, colsum, cw, Wa, Wh, Wc,
                      bs1.reshape(1, D), Ws2, bs2.reshape(1, 1))
    return scores.reshape(-1)[:NL]


# speedup vs baseline: 13.5930x; 1.2023x over previous
import functools

import jax
import jax.numpy as jnp
from jax import lax
from jax.experimental import pallas as pl
from jax.experimental.pallas import tpu as pltpu
from jax.experimental.pallas import tpu_sc as plsc

N = 10000
E = 160000
D = 256
HD = 128
NL = 5000
NLP = 5120
NP = 10240
NC = 2
NS = 16
EK = 40

_SC_MESH = dict(core_axis_name="c", subcore_axis_name="s",
                num_cores=NC, num_subcores=NS)


_DCH = 125
_NDCH = E // NC // NS // _DCH


def _sc_deg_body(dst4_hbm, degp_hbm, didx_d, onesv, bounce, deg_sh):
    s = lax.axis_index("s")
    c = lax.axis_index("c")

    def _zero16(i, carry):
        bounce[pl.ds(i * 16, 16)] = jnp.zeros((16,), jnp.float32)
        return carry

    lax.fori_loop(0, 640 // 16, _zero16, 0)
    pltpu.sync_copy(bounce, deg_sh.at[pl.ds(s * 640, 640)])
    pltpu.sync_copy(dst4_hbm.at[c, s], didx_d)

    def _ones16(i, carry):
        onesv[pl.ds(i * 16, 16)] = jnp.ones((16,), jnp.float32)
        return carry

    lax.fori_loop(0, _DCH // 16, _ones16, 0)
    onesv[pl.ds(_DCH - 16, 16)] = jnp.ones((16,), jnp.float32)
    plsc.subcore_barrier()

    def _step(j, carry):
        pltpu.sync_copy(onesv, deg_sh.at[didx_d.at[j]], add=True)
        return carry

    lax.fori_loop(0, _NDCH, _step, 0)
    plsc.subcore_barrier()

    pltpu.sync_copy(deg_sh.at[pl.ds(s * 640, 640)], bounce)
    pltpu.sync_copy(bounce, degp_hbm.at[c, pl.ds(s * 640, 640)])


_deg_call = pl.kernel(
    _sc_deg_body,
    out_type=jax.ShapeDtypeStruct((NC, NP), jnp.float32),
    mesh=plsc.VectorSubcoreMesh(**_SC_MESH),
    scratch_types=[
        pltpu.VMEM((_NDCH, _DCH), jnp.int32),
        pltpu.VMEM((_DCH,), jnp.float32),
        pltpu.VMEM((640,), jnp.float32),
        pltpu.VMEM_SHARED((NP,), jnp.float32),
    ],
)


EKA = 100


_NCH = (E // NS) // EKA


_NSUP = 4
_SUP = _NCH // _NSUP


def _sc_agg_body(hd_hbm, src4_hbm, dst4_hbm, S_hbm,
                 sidx_m, didx_m, rows0, rows1, zb,
                 sem0, sem1, acc_sh):
    s = lax.axis_index("s")
    c = lax.axis_index("c")
    rows = (rows0, rows1)
    sems = (sem0, sem1)

    def _zero16(i, carry):
        zb[i // 8, pl.ds((i % 8) * 16, 16)] = jnp.zeros((16,), jnp.float32)
        return carry

    lax.fori_loop(0, 16 * 8, _zero16, 0)

    def _zshared(j, carry):
        pltpu.sync_copy(zb, acc_sh.at[pl.ds(s * 640 + j * 16, 16)])
        return carry

    lax.fori_loop(0, 640 // 16, _zshared, 0)
    plsc.subcore_barrier()

    def _super(g, carry):
        pltpu.sync_copy(src4_hbm.at[s, g], sidx_m)
        pltpu.sync_copy(dst4_hbm.at[s, g], didx_m)
        pltpu.async_copy(hd_hbm.at[c].at[sidx_m.at[0]], rows0, sem0)

        def _pair(i, carry2):
            for b in range(2):
                k = i * 2 + b
                nxt = 1 - b
                pltpu.async_copy(hd_hbm.at[c].at[sidx_m.at[k + 1]],
                                 rows[nxt], sems[nxt])
                pltpu.make_async_copy(hd_hbm.at[c].at[sidx_m.at[k]], rows[b],
                                      sems[b]).wait()
                pltpu.sync_copy(rows[b], acc_sh.at[didx_m.at[k]], add=True)
            return carry2

        lax.fori_loop(0, (_SUP - 1) // 2, _pair, 0)
        pltpu.make_async_copy(hd_hbm.at[c].at[sidx_m.at[_SUP - 1]], rows0,
                              sem0).wait()
        pltpu.sync_copy(rows0, acc_sh.at[didx_m.at[_SUP - 1]], add=True)
        return carry

    lax.fori_loop(0, _NSUP, _super, 0)
    plsc.subcore_barrier()

    pltpu.sync_copy(acc_sh.at[pl.ds(s * 640, 640)],
                    S_hbm.at[c].at[pl.ds(s * 640, 640)])


_agg_call = pl.kernel(
    _sc_agg_body,
    out_type=jax.ShapeDtypeStruct((NC, NP, HD), jnp.float32),
    mesh=plsc.VectorSubcoreMesh(**_SC_MESH),
    scratch_types=[
        pltpu.VMEM((_SUP, EKA), jnp.int32),
        pltpu.VMEM((_SUP, EKA), jnp.int32),
        pltpu.VMEM((EKA, HD), jnp.float32),
        pltpu.VMEM((EKA, HD), jnp.float32),
        pltpu.VMEM((16, HD), jnp.float32),
        pltpu.SemaphoreType.DMA,
        pltpu.SemaphoreType.DMA,
        pltpu.VMEM_SHARED((NP, HD), jnp.float32),
    ],
)


def _sc_leaf_body(h2_hbm, li_hbm, out_hbm, lidx, lrows, sem):
    s = lax.axis_index("s")
    c = lax.axis_index("c")
    w = s * NC + c

    def _step(j, carry):
        base = w * 160 + j * 80
        pltpu.sync_copy(li_hbm.at[pl.ds(base, 80)], lidx)
        pltpu.async_copy(h2_hbm.at[lidx], lrows, sem).wait()
        pltpu.sync_copy(lrows, out_hbm.at[pl.ds(base, 80)])
        return carry

    lax.fori_loop(0, 2, _step, 0)


_leaf_call = pl.kernel(
    _sc_leaf_body,
    out_type=jax.ShapeDtypeStruct((NLP, D), jnp.float32),
    mesh=plsc.VectorSubcoreMesh(**_SC_MESH),
    scratch_types=[
        pltpu.VMEM((80,), jnp.int32),
        pltpu.VMEM((80, D), jnp.float32),
        pltpu.SemaphoreType.DMA,
    ],
)


_RB = 2000


def _tc_a_body(degp_ref, x_ref, w1_ref, hw_ref, hd_ref, dinv_ref, dinv2_ref):
    deg = degp_ref[0] + degp_ref[1] + 1.0
    dinv = lax.rsqrt(deg)
    dinv2 = dinv * dinv
    hw = jnp.dot(x_ref[...], w1_ref[...], preferred_element_type=jnp.float32)
    hw_ref[...] = hw
    hd_ref[0] = hw[:, :HD] * dinv
    hd_ref[1] = hw[:, HD:] * dinv
    dinv_ref[...] = dinv
    dinv2_ref[...] = dinv2


def _stage_a(degp, x, W1):
    return pl.pallas_call(
        _tc_a_body,
        grid=(N // _RB,),
        in_specs=[
            pl.BlockSpec((NC, _RB, 1), lambda i: (0, i, 0)),
            pl.BlockSpec((_RB, D), lambda i: (i, 0)),
            pl.BlockSpec((D, D), lambda i: (0, 0)),
        ],
        out_specs=[
            pl.BlockSpec((_RB, D), lambda i: (i, 0)),
            pl.BlockSpec((NC, _RB, HD), lambda i: (0, i, 0)),
            pl.BlockSpec((_RB, 1), lambda i: (i, 0)),
            pl.BlockSpec((_RB, 1), lambda i: (i, 0)),
        ],
        out_shape=[
            jax.ShapeDtypeStruct((N, D), jnp.float32),
            jax.ShapeDtypeStruct((NC, N, HD), jnp.float32),
            jax.ShapeDtypeStruct((N, 1), jnp.float32),
            jax.ShapeDtypeStruct((N, 1), jnp.float32),
        ],
    )(degp, x, W1)


def _tc_b_body(S_ref, hw_ref, dinv_ref, dinv2_ref, b_ref, w2_ref,
               hw2_ref, hd2_ref):
    Sc = jnp.concatenate([S_ref[0], S_ref[1]], axis=1)
    dinv = dinv_ref[...]
    h1 = Sc * dinv + hw_ref[...] * dinv2_ref[...] + b_ref[...]
    h1 = jnp.maximum(h1, 0.0)
    hw2 = jnp.dot(h1, w2_ref[...], preferred_element_type=jnp.float32)
    hw2_ref[...] = hw2
    hd2_ref[0] = hw2[:, :HD] * dinv
    hd2_ref[1] = hw2[:, HD:] * dinv


def _stage_b(S1, hw1, dinv, dinv2, b1, W2):
    return pl.pallas_call(
        _tc_b_body,
        grid=(N // _RB,),
        in_specs=[
            pl.BlockSpec((NC, _RB, HD), lambda i: (0, i, 0)),
            pl.BlockSpec((_RB, D), lambda i: (i, 0)),
            pl.BlockSpec((_RB, 1), lambda i: (i, 0)),
            pl.BlockSpec((_RB, 1), lambda i: (i, 0)),
            pl.BlockSpec((1, D), lambda i: (0, 0)),
            pl.BlockSpec((D, D), lambda i: (0, 0)),
        ],
        out_specs=[
            pl.BlockSpec((_RB, D), lambda i: (i, 0)),
            pl.BlockSpec((NC, _RB, HD), lambda i: (0, i, 0)),
        ],
        out_shape=[
            jax.ShapeDtypeStruct((N, D), jnp.float32),
            jax.ShapeDtypeStruct((NC, N, HD), jnp.float32),
        ],
    )(S1, hw1, dinv, dinv2, b1, W2)


def _tc_c_body(S_ref, hw_ref, dinv_ref, dinv2_ref, b_ref, h2_ref, cs_ref):
    i = pl.program_id(0)
    Sc = jnp.concatenate([S_ref[0], S_ref[1]], axis=1)
    h2 = Sc * dinv_ref[...] + hw_ref[...] * dinv2_ref[...] + b_ref[...]
    h2_ref[...] = h2

    @pl.when(i == 0)
    def _():
        cs_ref[...] = jnp.zeros_like(cs_ref)

    cs_ref[...] += jnp.sum(h2, axis=0, keepdims=True)


def _stage_c(S2, hw2, dinv, dinv2, b2):
    return pl.pallas_call(
        _tc_c_body,
        grid=(N // _RB,),
        in_specs=[
            pl.BlockSpec((NC, _RB, HD), lambda i: (0, i, 0)),
            pl.BlockSpec((_RB, D), lambda i: (i, 0)),
            pl.BlockSpec((_RB, 1), lambda i: (i, 0)),
            pl.BlockSpec((_RB, 1), lambda i: (i, 0)),
            pl.BlockSpec((1, D), lambda i: (0, 0)),
        ],
        out_specs=[
            pl.BlockSpec((_RB, D), lambda i: (i, 0)),
            pl.BlockSpec((1, D), lambda i: (0, 0)),
        ],
        out_shape=[
            jax.ShapeDtypeStruct((N, D), jnp.float32),
            jax.ShapeDtypeStruct((1, D), jnp.float32),
        ],
    )(S2, hw2, dinv, dinv2, b2)


def _tc_d_body(g_ref, cs_ref, cw_ref, wa_ref, wh_ref, wc_ref, bs1_ref,
               ws2_ref, bs2_ref, out_ref):
    gh = cs_ref[...] * (1.0 / N)
    c0 = (jnp.dot(cw_ref[...], wa_ref[...], preferred_element_type=jnp.float32)
          + jnp.dot(gh, wc_ref[...], preferred_element_type=jnp.float32)
          + bs1_ref[...])
    sact = jnp.dot(g_ref[...], wh_ref[...], preferred_element_type=jnp.float32)
    sact = jnp.maximum(sact + c0, 0.0)
    logits = (jnp.dot(sact, ws2_ref[...], preferred_element_type=jnp.float32)
              + bs2_ref[...])
    rows = lax.broadcasted_iota(jnp.int32, (NLP, 1), 0)
    logits = jnp.where(rows < NL, logits, -1e30)
    m = jnp.max(logits)
    p = jnp.exp(logits - m)
    out_ref[...] = p / jnp.sum(p)


def _stage_d(g, colsum, cw, Wa, Wh, Wc, bs1, Ws2, bs2):
    return pl.pallas_call(
        _tc_d_body,
        out_shape=jax.ShapeDtypeStruct((NLP, 1), jnp.float32),
    )(g, colsum, cw, Wa, Wh, Wc, bs1, Ws2, bs2)


def kernel(x, edge_index, chosen_wordemb, leave_inds, W1, b1, W2, b2,
           Ws1, bs1, Ws2, bs2):
    src = edge_index[0].astype(jnp.int32)
    dst = edge_index[1].astype(jnp.int32)
    li = jnp.pad(leave_inds.astype(jnp.int32), (0, NLP - NL))

    dst4 = dst.reshape(NC, NS, _NDCH, _DCH)
    degp = _deg_call(dst4)
    degp3 = degp[:, :N, None]

    src3 = src.reshape(NS, _NSUP, _SUP, EKA)
    dst3 = dst.reshape(NS, _NSUP, _SUP, EKA)
    hw1, hd1, dinv, dinv2 = _stage_a(degp3, x, W1)
    S1 = _agg_call(hd1, src3, dst3)
    hw2, hd2 = _stage_b(S1[:, :N], hw1, dinv, dinv2, b1.reshape(1, D), W2)
    S2 = _agg_call(hd2, src3, dst3)
    h2, colsum = _stage_c(S2[:, :N], hw2, dinv, dinv2, b2.reshape(1, D))

    g = _leaf_call(h2, li)

    cw = chosen_wordemb.reshape(1, D)
    Wa, Wh, Wc = Ws1[:D], Ws1[D:2 * D], Ws1[2 * D:]
    scores = _stage_d(g, colsum, cw, Wa, Wh, Wc,
                      bs1.reshape(1, D), Ws2, bs2.reshape(1, 1))
    return scores.reshape(-1)[:NL]


# speedup vs baseline: 13.7909x; 1.0146x over previous
import jax
import jax.numpy as jnp
from jax import lax
from jax.experimental import pallas as pl
from jax.experimental.pallas import tpu as pltpu
from jax.experimental.pallas import tpu_sc as plsc

N = 10000
E = 160000
D = 256
HD = 128
NL = 5000
NLP = 5120
NP = 10240
NC = 2
NS = 16

_SC_MESH = dict(core_axis_name="c", subcore_axis_name="s",
                num_cores=NC, num_subcores=NS)


_DCH = 125
_NDCH = E // NC // NS // _DCH


def _sc_deg_body(dst4_hbm, degp_hbm, didx_d, onesv, bounce, deg_sh):
    s = lax.axis_index("s")
    c = lax.axis_index("c")

    def _zero16(i, carry):
        bounce[pl.ds(i * 16, 16)] = jnp.zeros((16,), jnp.float32)
        return carry

    lax.fori_loop(0, 640 // 16, _zero16, 0)
    pltpu.sync_copy(bounce, deg_sh.at[pl.ds(s * 640, 640)])
    pltpu.sync_copy(dst4_hbm.at[c, s], didx_d)

    def _ones16(i, carry):
        onesv[pl.ds(i * 16, 16)] = jnp.ones((16,), jnp.float32)
        return carry

    lax.fori_loop(0, _DCH // 16, _ones16, 0)
    onesv[pl.ds(_DCH - 16, 16)] = jnp.ones((16,), jnp.float32)
    plsc.subcore_barrier()

    def _step(j, carry):
        pltpu.sync_copy(onesv, deg_sh.at[didx_d.at[j]], add=True)
        return carry

    lax.fori_loop(0, _NDCH, _step, 0)
    plsc.subcore_barrier()

    pltpu.sync_copy(deg_sh.at[pl.ds(s * 640, 640)], bounce)
    pltpu.sync_copy(bounce, degp_hbm.at[c, pl.ds(s * 640, 640)])


_deg_call = pl.kernel(
    _sc_deg_body,
    out_type=jax.ShapeDtypeStruct((NC, NP), jnp.float32),
    mesh=plsc.VectorSubcoreMesh(**_SC_MESH),
    scratch_types=[
        pltpu.VMEM((_NDCH, _DCH), jnp.int32),
        pltpu.VMEM((_DCH,), jnp.float32),
        pltpu.VMEM((640,), jnp.float32),
        pltpu.VMEM_SHARED((NP,), jnp.float32),
    ],
)


EKA = 100
_NCH = (E // NS) // EKA
_NSUP = 4
_SUP = _NCH // _NSUP


def _agg_edges(hd_hbm, src4_hbm, dst4_hbm, S_hbm,
               sidx_m, didx_m, rows, zb, sems, acc_sh, s, c):
    def _zero16(i, carry):
        zb[i // 8, pl.ds((i % 8) * 16, 16)] = jnp.zeros((16,), jnp.float32)
        return carry

    lax.fori_loop(0, 16 * 8, _zero16, 0)

    def _zshared(j, carry):
        pltpu.sync_copy(zb, acc_sh.at[pl.ds(s * 640 + j * 16, 16)])
        return carry

    lax.fori_loop(0, 640 // 16, _zshared, 0)
    plsc.subcore_barrier()

    def _super(g, carry):
        pltpu.sync_copy(src4_hbm.at[s, g], sidx_m)
        pltpu.sync_copy(dst4_hbm.at[s, g], didx_m)
        pltpu.async_copy(hd_hbm.at[c].at[sidx_m.at[0]], rows[0], sems[0])

        def _pair(i, carry2):
            for b in range(2):
                k = i * 2 + b
                nxt = 1 - b
                pltpu.async_copy(hd_hbm.at[c].at[sidx_m.at[k + 1]],
                                 rows[nxt], sems[nxt])
                pltpu.make_async_copy(hd_hbm.at[c].at[sidx_m.at[k]], rows[b],
                                      sems[b]).wait()
                pltpu.sync_copy(rows[b], acc_sh.at[didx_m.at[k]], add=True)
            return carry2

        lax.fori_loop(0, (_SUP - 1) // 2, _pair, 0)
        pltpu.make_async_copy(hd_hbm.at[c].at[sidx_m.at[_SUP - 1]], rows[0],
                              sems[0]).wait()
        pltpu.sync_copy(rows[0], acc_sh.at[didx_m.at[_SUP - 1]], add=True)
        return carry

    lax.fori_loop(0, _NSUP, _super, 0)
    plsc.subcore_barrier()

    pltpu.sync_copy(acc_sh.at[pl.ds(s * 640, 640)],
                    S_hbm.at[c].at[pl.ds(s * 640, 640)])


def _sc_agg_body(hd_hbm, src4_hbm, dst4_hbm, S_hbm,
                 sidx_m, didx_m, rows0, rows1, zb,
                 sem0, sem1, acc_sh):
    s = lax.axis_index("s")
    c = lax.axis_index("c")
    _agg_edges(hd_hbm, src4_hbm, dst4_hbm, S_hbm,
               sidx_m, didx_m, (rows0, rows1), zb, (sem0, sem1), acc_sh, s, c)


_agg_call = pl.kernel(
    _sc_agg_body,
    out_type=jax.ShapeDtypeStruct((NC, NP, HD), jnp.float32),
    mesh=plsc.VectorSubcoreMesh(**_SC_MESH),
    scratch_types=[
        pltpu.VMEM((_SUP, EKA), jnp.int32),
        pltpu.VMEM((_SUP, EKA), jnp.int32),
        pltpu.VMEM((EKA, HD), jnp.float32),
        pltpu.VMEM((EKA, HD), jnp.float32),
        pltpu.VMEM((16, HD), jnp.float32),
        pltpu.SemaphoreType.DMA,
        pltpu.SemaphoreType.DMA,
        pltpu.VMEM_SHARED((NP, HD), jnp.float32),
    ],
)


_LCH = 80
_NLCH = NLP // NS // _LCH


def _sc_agg_leaf_body(hd_hbm, src4_hbm, dst4_hbm, li3_hbm, dinv_hbm,
                      S_hbm, gS_hbm, gH_hbm, gd_hbm,
                      sidx_m, didx_m, rows0, rows1, zb, lidx, ldv,
                      sem0, sem1, acc_sh):
    s = lax.axis_index("s")
    c = lax.axis_index("c")
    _agg_edges(hd_hbm, src4_hbm, dst4_hbm, S_hbm,
               sidx_m, didx_m, (rows0, rows1), zb, (sem0, sem1), acc_sh, s, c)
    plsc.subcore_barrier()

    lrows = rows0.at[pl.ds(0, _LCH)]

    def _leaf(q, carry):
        base = s * (_NLCH * _LCH) + q * _LCH
        pltpu.sync_copy(li3_hbm.at[s, q], lidx)
        pltpu.async_copy(S_hbm.at[c].at[lidx], lrows, sem0).wait()
        pltpu.sync_copy(lrows, gS_hbm.at[c].at[pl.ds(base, _LCH)])
        pltpu.async_copy(hd_hbm.at[c].at[lidx], lrows, sem0).wait()
        pltpu.sync_copy(lrows, gH_hbm.at[c].at[pl.ds(base, _LCH)])

        @pl.when(c == 0)
        def _():
            pltpu.async_copy(dinv_hbm.at[lidx], ldv, sem1).wait()
            pltpu.sync_copy(ldv, gd_hbm.at[pl.ds(base, _LCH)])

        return carry

    lax.fori_loop(0, _NLCH, _leaf, 0)


_agg_leaf_call = pl.kernel(
    _sc_agg_leaf_body,
    out_type=[
        jax.ShapeDtypeStruct((NC, NP, HD), jnp.float32),
        jax.ShapeDtypeStruct((NC, NLP, HD), jnp.float32),
        jax.ShapeDtypeStruct((NC, NLP, HD), jnp.float32),
        jax.ShapeDtypeStruct((NLP,), jnp.float32),
    ],
    mesh=plsc.VectorSubcoreMesh(**_SC_MESH),
    scratch_types=[
        pltpu.VMEM((_SUP, EKA), jnp.int32),
        pltpu.VMEM((_SUP, EKA), jnp.int32),
        pltpu.VMEM((EKA, HD), jnp.float32),
        pltpu.VMEM((EKA, HD), jnp.float32),
        pltpu.VMEM((16, HD), jnp.float32),
        pltpu.VMEM((_LCH,), jnp.int32),
        pltpu.VMEM((_LCH,), jnp.float32),
        pltpu.SemaphoreType.DMA,
        pltpu.SemaphoreType.DMA,
        pltpu.VMEM_SHARED((NP, HD), jnp.float32),
    ],
)


_RB = 2000


def _tc_a_body(degp_ref, x_ref, w1_ref, hd_ref, dinv_ref):
    deg = degp_ref[0] + degp_ref[1] + 1.0
    dinv = lax.rsqrt(deg)
    hw = jnp.dot(x_ref[...], w1_ref[...], preferred_element_type=jnp.float32)
    hd_ref[0] = hw[:, :HD] * dinv
    hd_ref[1] = hw[:, HD:] * dinv
    dinv_ref[...] = dinv


def _stage_a(degp, x, W1):
    return pl.pallas_call(
        _tc_a_body,
        grid=(N // _RB,),
        in_specs=[
            pl.BlockSpec((NC, _RB, 1), lambda i: (0, i, 0)),
            pl.BlockSpec((_RB, D), lambda i: (i, 0)),
            pl.BlockSpec((D, D), lambda i: (0, 0)),
        ],
        out_specs=[
            pl.BlockSpec((NC, _RB, HD), lambda i: (0, i, 0)),
            pl.BlockSpec((_RB, 1), lambda i: (i, 0)),
        ],
        out_shape=[
            jax.ShapeDtypeStruct((NC, N, HD), jnp.float32),
            jax.ShapeDtypeStruct((N, 1), jnp.float32),
        ],
    )(degp, x, W1)


def _tc_b_body(S_ref, hd_ref, dinv_ref, b_ref, w2_ref, hd2_ref):
    dinv = dinv_ref[...]
    t = jnp.concatenate([S_ref[0] + hd_ref[0], S_ref[1] + hd_ref[1]], axis=1)
    h1 = jnp.maximum(t * dinv + b_ref[...], 0.0)
    hw2 = jnp.dot(h1, w2_ref[...], preferred_element_type=jnp.float32)
    hd2_ref[0] = hw2[:, :HD] * dinv
    hd2_ref[1] = hw2[:, HD:] * dinv


def _stage_b(S1, hd1, dinv, b1, W2):
    return pl.pallas_call(
        _tc_b_body,
        grid=(N // _RB,),
        in_specs=[
            pl.BlockSpec((NC, _RB, HD), lambda i: (0, i, 0)),
            pl.BlockSpec((NC, _RB, HD), lambda i: (0, i, 0)),
            pl.BlockSpec((_RB, 1), lambda i: (i, 0)),
            pl.BlockSpec((1, D), lambda i: (0, 0)),
            pl.BlockSpec((D, D), lambda i: (0, 0)),
        ],
        out_specs=[
            pl.BlockSpec((NC, _RB, HD), lambda i: (0, i, 0)),
        ],
        out_shape=[
            jax.ShapeDtypeStruct((NC, N, HD), jnp.float32),
        ],
    )(S1, hd1, dinv, b1, W2)[0]


_NBLK = N // _RB


def _tc_d_body(S_ref, hd_ref, dinv_ref, b2_ref, gS_ref, gH_ref, gd_ref,
               cw_ref, wa_ref, wh_ref, wc_ref, bs1_ref, ws2_ref, bs2_ref,
               out_ref, cs_ref):
    i = pl.program_id(0)

    @pl.when(i == 0)
    def _():
        cs_ref[...] = jnp.zeros_like(cs_ref)

    @pl.when(i < _NBLK)
    def _():
        t = jnp.concatenate([S_ref[0] + hd_ref[0], S_ref[1] + hd_ref[1]],
                            axis=1) * dinv_ref[...]
        cs_ref[...] += jnp.sum(t, axis=0, keepdims=True)

    @pl.when(i == _NBLK)
    def _():
        b2 = b2_ref[...]
        gh = cs_ref[...] * (1.0 / N) + b2
        h2l = (jnp.concatenate([gS_ref[0] + gH_ref[0], gS_ref[1] + gH_ref[1]],
                               axis=1) * gd_ref[...] + b2)
        c0 = (jnp.dot(cw_ref[...], wa_ref[...],
                      preferred_element_type=jnp.float32)
              + jnp.dot(gh, wc_ref[...], preferred_element_type=jnp.float32)
              + bs1_ref[...])
        sact = jnp.dot(h2l, wh_ref[...], preferred_element_type=jnp.float32)
        sact = jnp.maximum(sact + c0, 0.0)
        logits = (jnp.dot(sact, ws2_ref[...],
                          preferred_element_type=jnp.float32) + bs2_ref[...])
        rows = lax.broadcasted_iota(jnp.int32, (NLP, 1), 0)
        logits = jnp.where(rows < NL, logits, -1e30)
        m = jnp.max(logits)
        p = jnp.exp(logits - m)
        out_ref[...] = p / jnp.sum(p)


def _stage_d(S2, hd2, dinv, b2, gS, gH, gd, cw, Wa, Wh, Wc, bs1, Ws2, bs2):
    blk = lambda i: (0, jnp.minimum(i, _NBLK - 1), 0)
    rowblk = lambda i: (jnp.minimum(i, _NBLK - 1), 0)
    full3 = lambda i: (0, 0, 0)
    full2 = lambda i: (0, 0)
    return pl.pallas_call(
        _tc_d_body,
        grid=(_NBLK + 1,),
        in_specs=[
            pl.BlockSpec((NC, _RB, HD), blk),
            pl.BlockSpec((NC, _RB, HD), blk),
            pl.BlockSpec((_RB, 1), rowblk),
            pl.BlockSpec((1, D), full2),
            pl.BlockSpec((NC, NLP, HD), full3),
            pl.BlockSpec((NC, NLP, HD), full3),
            pl.BlockSpec((NLP, 1), full2),
            pl.BlockSpec((1, D), full2),
            pl.BlockSpec((D, D), full2),
            pl.BlockSpec((D, D), full2),
            pl.BlockSpec((D, D), full2),
            pl.BlockSpec((1, D), full2),
            pl.BlockSpec((D, 1), full2),
            pl.BlockSpec((1, 1), full2),
        ],
        out_specs=pl.BlockSpec((NLP, 1), full2),
        out_shape=jax.ShapeDtypeStruct((NLP, 1), jnp.float32),
        scratch_shapes=[pltpu.VMEM((1, D), jnp.float32)],
    )(S2, hd2, dinv, b2, gS, gH, gd, cw, Wa, Wh, Wc, bs1, Ws2, bs2)


def kernel(x, edge_index, chosen_wordemb, leave_inds, W1, b1, W2, b2,
           Ws1, bs1, Ws2, bs2):
    src = edge_index[0].astype(jnp.int32)
    dst = edge_index[1].astype(jnp.int32)
    li = jnp.pad(leave_inds.astype(jnp.int32), (0, NLP - NL))
    li3 = li.reshape(NS, _NLCH, _LCH)

    dst4 = dst.reshape(NC, NS, _NDCH, _DCH)
    degp = _deg_call(dst4)
    degp3 = degp[:, :N, None]

    src3 = src.reshape(NS, _NSUP, _SUP, EKA)
    dst3 = dst.reshape(NS, _NSUP, _SUP, EKA)
    hd1, dinv = _stage_a(degp3, x, W1)
    S1 = _agg_call(hd1, src3, dst3)
    hd2 = _stage_b(S1, hd1, dinv, b1.reshape(1, D), W2)
    S2, gS, gH, gd = _agg_leaf_call(hd2, src3, dst3, li3, dinv.reshape(-1))

    cw = chosen_wordemb.reshape(1, D)
    Wa, Wh, Wc = Ws1[:D], Ws1[D:2 * D], Ws1[2 * D:]
    scores = _stage_d(S2, hd2, dinv, b2.reshape(1, D), gS, gH,
                      gd.reshape(NLP, 1), cw, Wa, Wh, Wc,
                      bs1.reshape(1, D), Ws2, bs2.reshape(1, 1))
    return scores.reshape(-1)[:NL]


# speedup vs baseline: 13.8049x; 1.0010x over previous
import jax
import jax.numpy as jnp
from jax import lax
from jax.experimental import pallas as pl
from jax.experimental.pallas import tpu as pltpu
from jax.experimental.pallas import tpu_sc as plsc

N = 10000
E = 160000
D = 256
HD = 128
NL = 5000
NLP = 5120
NP = 10240
NC = 2
NS = 16

_SC_MESH = dict(core_axis_name="c", subcore_axis_name="s",
                num_cores=NC, num_subcores=NS)


_DCH = 125
_NDCH = E // NC // NS // _DCH


def _sc_deg_body(dst4_hbm, degp_hbm, didx_d, onesv, bounce, deg_sh):
    s = lax.axis_index("s")
    c = lax.axis_index("c")

    def _zero16(i, carry):
        bounce[pl.ds(i * 16, 16)] = jnp.zeros((16,), jnp.float32)
        return carry

    lax.fori_loop(0, 640 // 16, _zero16, 0)
    pltpu.sync_copy(bounce, deg_sh.at[pl.ds(s * 640, 640)])
    pltpu.sync_copy(dst4_hbm.at[c, s], didx_d)

    def _ones16(i, carry):
        onesv[pl.ds(i * 16, 16)] = jnp.ones((16,), jnp.float32)
        return carry

    lax.fori_loop(0, _DCH // 16, _ones16, 0)
    onesv[pl.ds(_DCH - 16, 16)] = jnp.ones((16,), jnp.float32)
    plsc.subcore_barrier()

    def _step(j, carry):
        pltpu.sync_copy(onesv, deg_sh.at[didx_d.at[j]], add=True)
        return carry

    lax.fori_loop(0, _NDCH, _step, 0)
    plsc.subcore_barrier()

    pltpu.sync_copy(deg_sh.at[pl.ds(s * 640, 640)], bounce)
    pltpu.sync_copy(bounce, degp_hbm.at[c, pl.ds(s * 640, 640)])


_deg_call = pl.kernel(
    _sc_deg_body,
    out_type=jax.ShapeDtypeStruct((NC, NP), jnp.float32),
    mesh=plsc.VectorSubcoreMesh(**_SC_MESH),
    scratch_types=[
        pltpu.VMEM((_NDCH, _DCH), jnp.int32),
        pltpu.VMEM((_DCH,), jnp.float32),
        pltpu.VMEM((640,), jnp.float32),
        pltpu.VMEM_SHARED((NP,), jnp.float32),
    ],
)


EKA = 100
_NCH = (E // NS) // EKA
_NSUP = 4
_SUP = _NCH // _NSUP


def _agg_edges(hd_hbm, src4_hbm, dst4_hbm, S_hbm,
               sidx_m, didx_m, rows, zb, sems, acc_sh, s, c):
    def _zero16(i, carry):
        zb[i // 8, pl.ds((i % 8) * 16, 16)] = jnp.zeros((16,), jnp.float32)
        return carry

    lax.fori_loop(0, 16 * 8, _zero16, 0)

    def _zshared(j, carry):
        pltpu.sync_copy(zb, acc_sh.at[pl.ds(s * 640 + j * 16, 16)])
        return carry

    lax.fori_loop(0, 640 // 16, _zshared, 0)
    plsc.subcore_barrier()

    gsems, ssems = sems

    def _gwait(k, b):
        pltpu.make_async_copy(hd_hbm.at[c].at[sidx_m.at[k]], rows[b],
                              gsems[b]).wait()

    def _swait(k, b):
        pltpu.make_async_copy(rows[b], acc_sh.at[didx_m.at[k]],
                              ssems[b]).wait()

    def _super(g, carry):
        pltpu.sync_copy(src4_hbm.at[s, g], sidx_m)
        pltpu.sync_copy(dst4_hbm.at[s, g], didx_m)
        pltpu.async_copy(hd_hbm.at[c].at[sidx_m.at[0]], rows[0], gsems[0])
        pltpu.async_copy(hd_hbm.at[c].at[sidx_m.at[1]], rows[1], gsems[1])
        _gwait(0, 0)
        pltpu.async_copy(rows[0], acc_sh.at[didx_m.at[0]], ssems[0], add=True)

        def _pair(i, carry2):
            for b in range(2):
                k = i * 2 + 1 + b
                bb = 1 - b
                nxt = 1 - bb
                _swait(k - 1, nxt)
                pltpu.async_copy(hd_hbm.at[c].at[sidx_m.at[k + 1]],
                                 rows[nxt], gsems[nxt])
                _gwait(k, bb)
                pltpu.async_copy(rows[bb], acc_sh.at[didx_m.at[k]],
                                 ssems[bb], add=True)
            return carry2

        lax.fori_loop(0, (_SUP - 3) // 2, _pair, 0)
        _swait(_SUP - 3, 0)
        pltpu.async_copy(hd_hbm.at[c].at[sidx_m.at[_SUP - 1]], rows[0],
                         gsems[0])
        _gwait(_SUP - 2, 1)
        pltpu.async_copy(rows[1], acc_sh.at[didx_m.at[_SUP - 2]], ssems[1],
                         add=True)
        _gwait(_SUP - 1, 0)
        pltpu.async_copy(rows[0], acc_sh.at[didx_m.at[_SUP - 1]], ssems[0],
                         add=True)
        _swait(_SUP - 2, 1)
        _swait(_SUP - 1, 0)
        return carry

    lax.fori_loop(0, _NSUP, _super, 0)
    plsc.subcore_barrier()

    pltpu.sync_copy(acc_sh.at[pl.ds(s * 640, 640)],
                    S_hbm.at[c].at[pl.ds(s * 640, 640)])


def _sc_agg_body(hd_hbm, src4_hbm, dst4_hbm, S_hbm,
                 sidx_m, didx_m, rows0, rows1, zb,
                 gsem0, gsem1, ssem0, ssem1, acc_sh):
    s = lax.axis_index("s")
    c = lax.axis_index("c")
    _agg_edges(hd_hbm, src4_hbm, dst4_hbm, S_hbm,
               sidx_m, didx_m, (rows0, rows1), zb,
               ((gsem0, gsem1), (ssem0, ssem1)), acc_sh, s, c)


_agg_call = pl.kernel(
    _sc_agg_body,
    out_type=jax.ShapeDtypeStruct((NC, NP, HD), jnp.float32),
    mesh=plsc.VectorSubcoreMesh(**_SC_MESH),
    scratch_types=[
        pltpu.VMEM((_SUP, EKA), jnp.int32),
        pltpu.VMEM((_SUP, EKA), jnp.int32),
        pltpu.VMEM((EKA, HD), jnp.float32),
        pltpu.VMEM((EKA, HD), jnp.float32),
        pltpu.VMEM((16, HD), jnp.float32),
        pltpu.SemaphoreType.DMA,
        pltpu.SemaphoreType.DMA,
        pltpu.SemaphoreType.DMA,
        pltpu.SemaphoreType.DMA,
        pltpu.VMEM_SHARED((NP, HD), jnp.float32),
    ],
)


_LCH = 80
_NLCH = NLP // NS // _LCH


def _sc_agg_leaf_body(hd_hbm, src4_hbm, dst4_hbm, li3_hbm, dinv_hbm,
                      S_hbm, gS_hbm, gH_hbm, gd_hbm,
                      sidx_m, didx_m, rows0, rows1, zb, lidx, ldv,
                      sem0, sem1, ssem0, ssem1, acc_sh):
    s = lax.axis_index("s")
    c = lax.axis_index("c")
    _agg_edges(hd_hbm, src4_hbm, dst4_hbm, S_hbm,
               sidx_m, didx_m, (rows0, rows1), zb,
               ((sem0, sem1), (ssem0, ssem1)), acc_sh, s, c)
    plsc.subcore_barrier()

    lrows = rows0.at[pl.ds(0, _LCH)]

    def _leaf(q, carry):
        base = s * (_NLCH * _LCH) + q * _LCH
        pltpu.sync_copy(li3_hbm.at[s, q], lidx)
        pltpu.async_copy(S_hbm.at[c].at[lidx], lrows, sem0).wait()
        pltpu.sync_copy(lrows, gS_hbm.at[c].at[pl.ds(base, _LCH)])
        pltpu.async_copy(hd_hbm.at[c].at[lidx], lrows, sem0).wait()
        pltpu.sync_copy(lrows, gH_hbm.at[c].at[pl.ds(base, _LCH)])

        @pl.when(c == 0)
        def _():
            pltpu.async_copy(dinv_hbm.at[lidx], ldv, sem1).wait()
            pltpu.sync_copy(ldv, gd_hbm.at[pl.ds(base, _LCH)])

        return carry

    lax.fori_loop(0, _NLCH, _leaf, 0)


_agg_leaf_call = pl.kernel(
    _sc_agg_leaf_body,
    out_type=[
        jax.ShapeDtypeStruct((NC, NP, HD), jnp.float32),
        jax.ShapeDtypeStruct((NC, NLP, HD), jnp.float32),
        jax.ShapeDtypeStruct((NC, NLP, HD), jnp.float32),
        jax.ShapeDtypeStruct((NLP,), jnp.float32),
    ],
    mesh=plsc.VectorSubcoreMesh(**_SC_MESH),
    scratch_types=[
        pltpu.VMEM((_SUP, EKA), jnp.int32),
        pltpu.VMEM((_SUP, EKA), jnp.int32),
        pltpu.VMEM((EKA, HD), jnp.float32),
        pltpu.VMEM((EKA, HD), jnp.float32),
        pltpu.VMEM((16, HD), jnp.float32),
        pltpu.VMEM((_LCH,), jnp.int32),
        pltpu.VMEM((_LCH,), jnp.float32),
        pltpu.SemaphoreType.DMA,
        pltpu.SemaphoreType.DMA,
        pltpu.SemaphoreType.DMA,
        pltpu.SemaphoreType.DMA,
        pltpu.VMEM_SHARED((NP, HD), jnp.float32),
    ],
)


_RB = 2000


def _tc_a_body(degp_ref, x_ref, w1_ref, hd_ref, dinv_ref):
    deg = degp_ref[0] + degp_ref[1] + 1.0
    dinv = lax.rsqrt(deg)
    hw = jnp.dot(x_ref[...], w1_ref[...], preferred_element_type=jnp.float32)
    hd_ref[0] = hw[:, :HD] * dinv
    hd_ref[1] = hw[:, HD:] * dinv
    dinv_ref[...] = dinv


def _stage_a(degp, x, W1):
    return pl.pallas_call(
        _tc_a_body,
        grid=(N // _RB,),
        in_specs=[
            pl.BlockSpec((NC, _RB, 1), lambda i: (0, i, 0)),
            pl.BlockSpec((_RB, D), lambda i: (i, 0)),
            pl.BlockSpec((D, D), lambda i: (0, 0)),
        ],
        out_specs=[
            pl.BlockSpec((NC, _RB, HD), lambda i: (0, i, 0)),
            pl.BlockSpec((_RB, 1), lambda i: (i, 0)),
        ],
        out_shape=[
            jax.ShapeDtypeStruct((NC, N, HD), jnp.float32),
            jax.ShapeDtypeStruct((N, 1), jnp.float32),
        ],
    )(degp, x, W1)


def _tc_b_body(S_ref, hd_ref, dinv_ref, b_ref, w2_ref, hd2_ref):
    dinv = dinv_ref[...]
    t = jnp.concatenate([S_ref[0] + hd_ref[0], S_ref[1] + hd_ref[1]], axis=1)
    h1 = jnp.maximum(t * dinv + b_ref[...], 0.0)
    hw2 = jnp.dot(h1, w2_ref[...], preferred_element_type=jnp.float32)
    hd2_ref[0] = hw2[:, :HD] * dinv
    hd2_ref[1] = hw2[:, HD:] * dinv


def _stage_b(S1, hd1, dinv, b1, W2):
    return pl.pallas_call(
        _tc_b_body,
        grid=(N // _RB,),
        in_specs=[
            pl.BlockSpec((NC, _RB, HD), lambda i: (0, i, 0)),
            pl.BlockSpec((NC, _RB, HD), lambda i: (0, i, 0)),
            pl.BlockSpec((_RB, 1), lambda i: (i, 0)),
            pl.BlockSpec((1, D), lambda i: (0, 0)),
            pl.BlockSpec((D, D), lambda i: (0, 0)),
        ],
        out_specs=[
            pl.BlockSpec((NC, _RB, HD), lambda i: (0, i, 0)),
        ],
        out_shape=[
            jax.ShapeDtypeStruct((NC, N, HD), jnp.float32),
        ],
    )(S1, hd1, dinv, b1, W2)[0]


_NBLK = N // _RB


def _tc_d_body(S_ref, hd_ref, dinv_ref, b2_ref, gS_ref, gH_ref, gd_ref,
               cw_ref, wa_ref, wh_ref, wc_ref, bs1_ref, ws2_ref, bs2_ref,
               out_ref, cs_ref):
    i = pl.program_id(0)

    @pl.when(i == 0)
    def _():
        cs_ref[...] = jnp.zeros_like(cs_ref)

    @pl.when(i < _NBLK)
    def _():
        t = jnp.concatenate([S_ref[0] + hd_ref[0], S_ref[1] + hd_ref[1]],
                            axis=1) * dinv_ref[...]
        cs_ref[...] += jnp.sum(t, axis=0, keepdims=True)

    @pl.when(i == _NBLK)
    def _():
        b2 = b2_ref[...]
        gh = cs_ref[...] * (1.0 / N) + b2
        h2l = (jnp.concatenate([gS_ref[0] + gH_ref[0], gS_ref[1] + gH_ref[1]],
                               axis=1) * gd_ref[...] + b2)
        c0 = (jnp.dot(cw_ref[...], wa_ref[...],
                      preferred_element_type=jnp.float32)
              + jnp.dot(gh, wc_ref[...], preferred_element_type=jnp.float32)
              + bs1_ref[...])
        sact = jnp.dot(h2l, wh_ref[...], preferred_element_type=jnp.float32)
        sact = jnp.maximum(sact + c0, 0.0)
        logits = (jnp.dot(sact, ws2_ref[...],
                          preferred_element_type=jnp.float32) + bs2_ref[...])
        rows = lax.broadcasted_iota(jnp.int32, (NLP, 1), 0)
        logits = jnp.where(rows < NL, logits, -1e30)
        m = jnp.max(logits)
        p = jnp.exp(logits - m)
        out_ref[...] = p / jnp.sum(p)


def _stage_d(S2, hd2, dinv, b2, gS, gH, gd, cw, Wa, Wh, Wc, bs1, Ws2, bs2):
    blk = lambda i: (0, jnp.minimum(i, _NBLK - 1), 0)
    rowblk = lambda i: (jnp.minimum(i, _NBLK - 1), 0)
    full3 = lambda i: (0, 0, 0)
    full2 = lambda i: (0, 0)
    return pl.pallas_call(
        _tc_d_body,
        grid=(_NBLK + 1,),
        in_specs=[
            pl.BlockSpec((NC, _RB, HD), blk),
            pl.BlockSpec((NC, _RB, HD), blk),
            pl.BlockSpec((_RB, 1), rowblk),
            pl.BlockSpec((1, D), full2),
            pl.BlockSpec((NC, NLP, HD), full3),
            pl.BlockSpec((NC, NLP, HD), full3),
            pl.BlockSpec((NLP, 1), full2),
            pl.BlockSpec((1, D), full2),
            pl.BlockSpec((D, D), full2),
            pl.BlockSpec((D, D), full2),
            pl.BlockSpec((D, D), full2),
            pl.BlockSpec((1, D), full2),
            pl.BlockSpec((D, 1), full2),
            pl.BlockSpec((1, 1), full2),
        ],
        out_specs=pl.BlockSpec((NLP, 1), full2),
        out_shape=jax.ShapeDtypeStruct((NLP, 1), jnp.float32),
        scratch_shapes=[pltpu.VMEM((1, D), jnp.float32)],
    )(S2, hd2, dinv, b2, gS, gH, gd, cw, Wa, Wh, Wc, bs1, Ws2, bs2)


def kernel(x, edge_index, chosen_wordemb, leave_inds, W1, b1, W2, b2,
           Ws1, bs1, Ws2, bs2):
    src = edge_index[0].astype(jnp.int32)
    dst = edge_index[1].astype(jnp.int32)
    li = jnp.pad(leave_inds.astype(jnp.int32), (0, NLP - NL))
    li3 = li.reshape(NS, _NLCH, _LCH)

    dst4 = dst.reshape(NC, NS, _NDCH, _DCH)
    degp = _deg_call(dst4)
    degp3 = degp[:, :N, None]

    src3 = src.reshape(NS, _NSUP, _SUP, EKA)
    dst3 = dst.reshape(NS, _NSUP, _SUP, EKA)
    hd1, dinv = _stage_a(degp3, x, W1)
    S1 = _agg_call(hd1, src3, dst3)
    hd2 = _stage_b(S1, hd1, dinv, b1.reshape(1, D), W2)
    S2, gS, gH, gd = _agg_leaf_call(hd2, src3, dst3, li3, dinv.reshape(-1))

    cw = chosen_wordemb.reshape(1, D)
    Wa, Wh, Wc = Ws1[:D], Ws1[D:2 * D], Ws1[2 * D:]
    scores = _stage_d(S2, hd2, dinv, b2.reshape(1, D), gS, gH,
                      gd.reshape(NLP, 1), cw, Wa, Wh, Wc,
                      bs1.reshape(1, D), Ws2, bs2.reshape(1, 1))
    return scores.reshape(-1)[:NL]


# speedup vs baseline: 13.8393x; 1.0025x over previous
import jax
import jax.numpy as jnp
from jax import lax
from jax.experimental import pallas as pl
from jax.experimental.pallas import tpu as pltpu
from jax.experimental.pallas import tpu_sc as plsc

N = 10000
E = 160000
D = 256
HD = 128
NL = 5000
NLP = 5120
NP = 10240
NC = 2
NS = 16

_SC_MESH = dict(core_axis_name="c", subcore_axis_name="s",
                num_cores=NC, num_subcores=NS)


_DCH = 125
_NDCH = E // NC // NS // _DCH


def _sc_deg_body(dst4_hbm, degp_hbm, didx_d, onesv, bounce, deg_sh):
    s = lax.axis_index("s")
    c = lax.axis_index("c")

    def _zero16(i, carry):
        bounce[pl.ds(i * 16, 16)] = jnp.zeros((16,), jnp.float32)
        return carry

    lax.fori_loop(0, 640 // 16, _zero16, 0)
    pltpu.sync_copy(bounce, deg_sh.at[pl.ds(s * 640, 640)])
    pltpu.sync_copy(dst4_hbm.at[c, s], didx_d)

    def _ones16(i, carry):
        onesv[pl.ds(i * 16, 16)] = jnp.ones((16,), jnp.float32)
        return carry

    lax.fori_loop(0, _DCH // 16, _ones16, 0)
    onesv[pl.ds(_DCH - 16, 16)] = jnp.ones((16,), jnp.float32)
    plsc.subcore_barrier()

    def _step(j, carry):
        pltpu.sync_copy(onesv, deg_sh.at[didx_d.at[j]], add=True)
        return carry

    lax.fori_loop(0, _NDCH, _step, 0)
    plsc.subcore_barrier()

    pltpu.sync_copy(deg_sh.at[pl.ds(s * 640, 640)], bounce)
    pltpu.sync_copy(bounce, degp_hbm.at[c, pl.ds(s * 640, 640)])


_deg_call = pl.kernel(
    _sc_deg_body,
    out_type=jax.ShapeDtypeStruct((NC, NP), jnp.float32),
    mesh=plsc.VectorSubcoreMesh(**_SC_MESH),
    scratch_types=[
        pltpu.VMEM((_NDCH, _DCH), jnp.int32),
        pltpu.VMEM((_DCH,), jnp.float32),
        pltpu.VMEM((640,), jnp.float32),
        pltpu.VMEM_SHARED((NP,), jnp.float32),
    ],
)


EKA = 100
_NCH = (E // NS) // EKA
_NSUP = 4
_SUP = _NCH // _NSUP


def _agg_edges(hd_hbm, src4_hbm, dst4_hbm, S_hbm,
               sidx_m, didx_m, rows, zb, sems, acc_sh, s, c):
    def _zero16(i, carry):
        zb[i // 8, pl.ds((i % 8) * 16, 16)] = jnp.zeros((16,), jnp.float32)
        return carry

    lax.fori_loop(0, 16 * 8, _zero16, 0)

    def _zshared(j, carry):
        pltpu.sync_copy(zb, acc_sh.at[pl.ds(s * 640 + j * 16, 16)])
        return carry

    lax.fori_loop(0, 640 // 16, _zshared, 0)
    plsc.subcore_barrier()

    gsems, ssems = sems

    def _gwait(k, b):
        pltpu.make_async_copy(hd_hbm.at[c].at[sidx_m.at[k]], rows[b],
                              gsems[b]).wait()

    def _swait(k, b):
        pltpu.make_async_copy(rows[b], acc_sh.at[didx_m.at[k]],
                              ssems[b]).wait()

    def _super(g, carry):
        pltpu.sync_copy(src4_hbm.at[s, g], sidx_m)
        pltpu.sync_copy(dst4_hbm.at[s, g], didx_m)
        pltpu.async_copy(hd_hbm.at[c].at[sidx_m.at[0]], rows[0], gsems[0])
        pltpu.async_copy(hd_hbm.at[c].at[sidx_m.at[1]], rows[1], gsems[1])
        _gwait(0, 0)
        pltpu.async_copy(rows[0], acc_sh.at[didx_m.at[0]], ssems[0], add=True)

        def _pair(i, carry2):
            for b in range(2):
                k = i * 2 + 1 + b
                bb = 1 - b
                nxt = 1 - bb
                _swait(k - 1, nxt)
                pltpu.async_copy(hd_hbm.at[c].at[sidx_m.at[k + 1]],
                                 rows[nxt], gsems[nxt])
                _gwait(k, bb)
                pltpu.async_copy(rows[bb], acc_sh.at[didx_m.at[k]],
                                 ssems[bb], add=True)
            return carry2

        lax.fori_loop(0, (_SUP - 3) // 2, _pair, 0)
        _swait(_SUP - 3, 0)
        pltpu.async_copy(hd_hbm.at[c].at[sidx_m.at[_SUP - 1]], rows[0],
                         gsems[0])
        _gwait(_SUP - 2, 1)
        pltpu.async_copy(rows[1], acc_sh.at[didx_m.at[_SUP - 2]], ssems[1],
                         add=True)
        _gwait(_SUP - 1, 0)
        pltpu.async_copy(rows[0], acc_sh.at[didx_m.at[_SUP - 1]], ssems[0],
                         add=True)
        _swait(_SUP - 2, 1)
        _swait(_SUP - 1, 0)
        return carry

    lax.fori_loop(0, _NSUP, _super, 0)
    plsc.subcore_barrier()

    pltpu.sync_copy(acc_sh.at[pl.ds(s * 640, 640)],
                    S_hbm.at[c].at[pl.ds(s * 640, 640)])


def _sc_agg_body(hd_hbm, src4_hbm, dst4_hbm, S_hbm,
                 sidx_m, didx_m, rows0, rows1, zb,
                 gsem0, gsem1, ssem0, ssem1, acc_sh):
    s = lax.axis_index("s")
    c = lax.axis_index("c")
    _agg_edges(hd_hbm, src4_hbm, dst4_hbm, S_hbm,
               sidx_m, didx_m, (rows0, rows1), zb,
               ((gsem0, gsem1), (ssem0, ssem1)), acc_sh, s, c)


_agg_call = pl.kernel(
    _sc_agg_body,
    out_type=jax.ShapeDtypeStruct((NC, NP, HD), jnp.float32),
    mesh=plsc.VectorSubcoreMesh(**_SC_MESH),
    scratch_types=[
        pltpu.VMEM((_SUP, EKA), jnp.int32),
        pltpu.VMEM((_SUP, EKA), jnp.int32),
        pltpu.VMEM((EKA, HD), jnp.float32),
        pltpu.VMEM((EKA, HD), jnp.float32),
        pltpu.VMEM((16, HD), jnp.float32),
        pltpu.SemaphoreType.DMA,
        pltpu.SemaphoreType.DMA,
        pltpu.SemaphoreType.DMA,
        pltpu.SemaphoreType.DMA,
        pltpu.VMEM_SHARED((NP, HD), jnp.float32),
    ],
)


_LCH = 80
_NLCH = NLP // NS // _LCH


def _sc_agg_leaf_body(hd_hbm, src4_hbm, dst4_hbm, li3_hbm, dinv_hbm,
                      S_hbm, gS_hbm, gH_hbm, gd_hbm,
                      sidx_m, didx_m, rows0, rows1, zb, lidx, ldv,
                      sem0, sem1, ssem0, ssem1, acc_sh):
    s = lax.axis_index("s")
    c = lax.axis_index("c")
    _agg_edges(hd_hbm, src4_hbm, dst4_hbm, S_hbm,
               sidx_m, didx_m, (rows0, rows1), zb,
               ((sem0, sem1), (ssem0, ssem1)), acc_sh, s, c)
    plsc.subcore_barrier()

    lrows = rows0.at[pl.ds(0, _LCH)]

    def _leaf(q, carry):
        base = s * (_NLCH * _LCH) + q * _LCH
        pltpu.sync_copy(li3_hbm.at[s, q], lidx)
        pltpu.async_copy(S_hbm.at[c].at[lidx], lrows, sem0).wait()
        pltpu.sync_copy(lrows, gS_hbm.at[c].at[pl.ds(base, _LCH)])
        pltpu.async_copy(hd_hbm.at[c].at[lidx], lrows, sem0).wait()
        pltpu.sync_copy(lrows, gH_hbm.at[c].at[pl.ds(base, _LCH)])

        @pl.when(c == lax.rem(q, 2))
        def _():
            pltpu.async_copy(dinv_hbm.at[lidx], ldv, sem1).wait()
            pltpu.sync_copy(ldv, gd_hbm.at[pl.ds(base, _LCH)])

        return carry

    lax.fori_loop(0, _NLCH, _leaf, 0)


_agg_leaf_call = pl.kernel(
    _sc_agg_leaf_body,
    out_type=[
        jax.ShapeDtypeStruct((NC, NP, HD), jnp.float32),
        jax.ShapeDtypeStruct((NC, NLP, HD), jnp.float32),
        jax.ShapeDtypeStruct((NC, NLP, HD), jnp.float32),
        jax.ShapeDtypeStruct((NLP,), jnp.float32),
    ],
    mesh=plsc.VectorSubcoreMesh(**_SC_MESH),
    scratch_types=[
        pltpu.VMEM((_SUP, EKA), jnp.int32),
        pltpu.VMEM((_SUP, EKA), jnp.int32),
        pltpu.VMEM((EKA, HD), jnp.float32),
        pltpu.VMEM((EKA, HD), jnp.float32),
        pltpu.VMEM((16, HD), jnp.float32),
        pltpu.VMEM((_LCH,), jnp.int32),
        pltpu.VMEM((_LCH,), jnp.float32),
        pltpu.SemaphoreType.DMA,
        pltpu.SemaphoreType.DMA,
        pltpu.SemaphoreType.DMA,
        pltpu.SemaphoreType.DMA,
        pltpu.VMEM_SHARED((NP, HD), jnp.float32),
    ],
)


_RB = 2000


def _tc_a_body(degp_ref, x_ref, w1_ref, hd_ref, dinv_ref):
    deg = degp_ref[0] + degp_ref[1] + 1.0
    dinv = lax.rsqrt(deg)
    hw = jnp.dot(x_ref[...], w1_ref[...], preferred_element_type=jnp.float32)
    hd_ref[0] = hw[:, :HD] * dinv
    hd_ref[1] = hw[:, HD:] * dinv
    dinv_ref[...] = dinv


def _stage_a(degp, x, W1):
    return pl.pallas_call(
        _tc_a_body,
        grid=(N // _RB,),
        in_specs=[
            pl.BlockSpec((NC, _RB, 1), lambda i: (0, i, 0)),
            pl.BlockSpec((_RB, D), lambda i: (i, 0)),
            pl.BlockSpec((D, D), lambda i: (0, 0)),
        ],
        out_specs=[
            pl.BlockSpec((NC, _RB, HD), lambda i: (0, i, 0)),
            pl.BlockSpec((_RB, 1), lambda i: (i, 0)),
        ],
        out_shape=[
            jax.ShapeDtypeStruct((NC, N, HD), jnp.float32),
            jax.ShapeDtypeStruct((N, 1), jnp.float32),
        ],
    )(degp, x, W1)


def _tc_b_body(S_ref, hd_ref, dinv_ref, b_ref, w2_ref, hd2_ref):
    dinv = dinv_ref[...]
    t = jnp.concatenate([S_ref[0] + hd_ref[0], S_ref[1] + hd_ref[1]], axis=1)
    h1 = jnp.maximum(t * dinv + b_ref[...], 0.0)
    hw2 = jnp.dot(h1, w2_ref[...], preferred_element_type=jnp.float32)
    hd2_ref[0] = hw2[:, :HD] * dinv
    hd2_ref[1] = hw2[:, HD:] * dinv


def _stage_b(S1, hd1, dinv, b1, W2):
    return pl.pallas_call(
        _tc_b_body,
        grid=(N // _RB,),
        in_specs=[
            pl.BlockSpec((NC, _RB, HD), lambda i: (0, i, 0)),
            pl.BlockSpec((NC, _RB, HD), lambda i: (0, i, 0)),
            pl.BlockSpec((_RB, 1), lambda i: (i, 0)),
            pl.BlockSpec((1, D), lambda i: (0, 0)),
            pl.BlockSpec((D, D), lambda i: (0, 0)),
        ],
        out_specs=[
            pl.BlockSpec((NC, _RB, HD), lambda i: (0, i, 0)),
        ],
        out_shape=[
            jax.ShapeDtypeStruct((NC, N, HD), jnp.float32),
        ],
    )(S1, hd1, dinv, b1, W2)[0]


_NBLK = N // _RB


def _tc_d_body(S_ref, hd_ref, dinv_ref, b2_ref, gS_ref, gH_ref, gd_ref,
               cw_ref, wa_ref, wh_ref, wc_ref, bs1_ref, ws2_ref, bs2_ref,
               out_ref, cs_ref):
    i = pl.program_id(0)

    @pl.when(i == 0)
    def _():
        cs_ref[...] = jnp.zeros_like(cs_ref)

    @pl.when(i < _NBLK)
    def _():
        t = jnp.concatenate([S_ref[0] + hd_ref[0], S_ref[1] + hd_ref[1]],
                            axis=1) * dinv_ref[...]
        cs_ref[...] += jnp.sum(t, axis=0, keepdims=True)

    @pl.when(i == _NBLK)
    def _():
        b2 = b2_ref[...]
        gh = cs_ref[...] * (1.0 / N) + b2
        h2l = (jnp.concatenate([gS_ref[0] + gH_ref[0], gS_ref[1] + gH_ref[1]],
                               axis=1) * gd_ref[...] + b2)
        c0 = (jnp.dot(cw_ref[...], wa_ref[...],
                      preferred_element_type=jnp.float32)
              + jnp.dot(gh, wc_ref[...], preferred_element_type=jnp.float32)
              + bs1_ref[...])
        sact = jnp.dot(h2l, wh_ref[...], preferred_element_type=jnp.float32)
        sact = jnp.maximum(sact + c0, 0.0)
        logits = (jnp.dot(sact, ws2_ref[...],
                          preferred_element_type=jnp.float32) + bs2_ref[...])
        rows = lax.broadcasted_iota(jnp.int32, (NLP, 1), 0)
        logits = jnp.where(rows < NL, logits, -1e30)
        m = jnp.max(logits)
        p = jnp.exp(logits - m)
        out_ref[...] = p / jnp.sum(p)


def _stage_d(S2, hd2, dinv, b2, gS, gH, gd, cw, Wa, Wh, Wc, bs1, Ws2, bs2):
    blk = lambda i: (0, jnp.minimum(i, _NBLK - 1), 0)
    rowblk = lambda i: (jnp.minimum(i, _NBLK - 1), 0)
    full3 = lambda i: (0, 0, 0)
    full2 = lambda i: (0, 0)
    return pl.pallas_call(
        _tc_d_body,
        grid=(_NBLK + 1,),
        in_specs=[
            pl.BlockSpec((NC, _RB, HD), blk),
            pl.BlockSpec((NC, _RB, HD), blk),
            pl.BlockSpec((_RB, 1), rowblk),
            pl.BlockSpec((1, D), full2),
            pl.BlockSpec((NC, NLP, HD), full3),
            pl.BlockSpec((NC, NLP, HD), full3),
            pl.BlockSpec((NLP, 1), full2),
            pl.BlockSpec((1, D), full2),
            pl.BlockSpec((D, D), full2),
            pl.BlockSpec((D, D), full2),
            pl.BlockSpec((D, D), full2),
            pl.BlockSpec((1, D), full2),
            pl.BlockSpec((D, 1), full2),
            pl.BlockSpec((1, 1), full2),
        ],
        out_specs=pl.BlockSpec((NLP, 1), full2),
        out_shape=jax.ShapeDtypeStruct((NLP, 1), jnp.float32),
        scratch_shapes=[pltpu.VMEM((1, D), jnp.float32)],
    )(S2, hd2, dinv, b2, gS, gH, gd, cw, Wa, Wh, Wc, bs1, Ws2, bs2)


def kernel(x, edge_index, chosen_wordemb, leave_inds, W1, b1, W2, b2,
           Ws1, bs1, Ws2, bs2):
    src = edge_index[0].astype(jnp.int32)
    dst = edge_index[1].astype(jnp.int32)
    li = jnp.pad(leave_inds.astype(jnp.int32), (0, NLP - NL))
    li3 = li.reshape(NS, _NLCH, _LCH)

    dst4 = dst.reshape(NC, NS, _NDCH, _DCH)
    degp = _deg_call(dst4)
    degp3 = degp[:, :N, None]

    src3 = src.reshape(NS, _NSUP, _SUP, EKA)
    dst3 = dst.reshape(NS, _NSUP, _SUP, EKA)
    hd1, dinv = _stage_a(degp3, x, W1)
    S1 = _agg_call(hd1, src3, dst3)
    hd2 = _stage_b(S1, hd1, dinv, b1.reshape(1, D), W2)
    S2, gS, gH, gd = _agg_leaf_call(hd2, src3, dst3, li3, dinv.reshape(-1))

    cw = chosen_wordemb.reshape(1, D)
    Wa, Wh, Wc = Ws1[:D], Ws1[D:2 * D], Ws1[2 * D:]
    scores = _stage_d(S2, hd2, dinv, b2.reshape(1, D), gS, gH,
                      gd.reshape(NLP, 1), cw, Wa, Wh, Wc,
                      bs1.reshape(1, D), Ws2, bs2.reshape(1, 1))
    return scores.reshape(-1)[:NL]
